# Initial kernel scaffold; baseline (speedup 1.0000x reference)
import jax
import jax.numpy as jnp
from jax.experimental import pallas as pl


def kernel(x, edge_index, W1, b1, W2, b2, Wb, bb):
    raise NotImplementedError("write your pallas kernel here")


# speedup vs baseline: 7.2850x; 7.2850x over previous
import functools

import jax
import jax.numpy as jnp
from jax import lax
from jax.experimental import pallas as pl
from jax.experimental.pallas import tpu as pltpu
from jax.experimental.pallas import tpu_sc as plsc

N = 10000
E = 320000
IN_CH = 128
HID = 128
OUT_CH = 64

NC = 2
NS = 16
NW = NC * NS
EPW = E // NW
CH = 80
NCHUNK = EPW // CH
ROWB = 80
NROWCH = N // ROWB

_MESH = plsc.VectorSubcoreMesh(
    core_axis_name="c", subcore_axis_name="s", num_cores=NC, num_subcores=NS)


def _fill(buf, rows, width, value):
  vec = jnp.full((16,), value, jnp.float32)

  def body(r, carry):
    for j in range(width // 16):
      buf[r, pl.ds(j * 16, 16)] = vec
    return carry

  lax.fori_loop(0, rows, body, 0)


def _sc_deg(dst):
  @functools.partial(
      pl.kernel,
      out_type=jax.ShapeDtypeStruct((NC, N, 16), jnp.float32),
      mesh=_MESH,
      compiler_params=pltpu.CompilerParams(use_tc_tiling_on_sc=False),
      scratch_types=[
          pltpu.VMEM((CH,), jnp.int32),
          pltpu.VMEM((ROWB, 16), jnp.float32),
          pltpu.VMEM_SHARED((N, 16), jnp.float32),
      ],
  )
  def k(dst_hbm, out_hbm, didx, rows, acc):
    c = lax.axis_index("c")
    s = lax.axis_index("s")
    gid = c * NS + s

    _fill(rows, ROWB, 16, 0.0)

    def zacc(j, carry):
      @pl.when(lax.rem(j, NS) == s)
      def _():
        pltpu.sync_copy(rows, acc.at[pl.ds(j * ROWB, ROWB)])
      return carry

    lax.fori_loop(0, NROWCH, zacc, 0)
    plsc.subcore_barrier()

    _fill(rows, ROWB, 16, 1.0)
    base = gid * EPW

    def step(j, carry):
      off = pl.multiple_of(base + j * CH, 8)
      pltpu.sync_copy(dst_hbm.at[pl.ds(off, CH)], didx)
      pltpu.sync_copy(rows, acc.at[didx], add=True)
      return carry

    lax.fori_loop(0, NCHUNK, step, 0)
    plsc.subcore_barrier()

    def cpout(j, carry):
      @pl.when(lax.rem(j, NS) == s)
      def _():
        pltpu.sync_copy(acc.at[pl.ds(j * ROWB, ROWB)], rows)
        pltpu.sync_copy(rows, out_hbm.at[c, pl.ds(j * ROWB, ROWB)])
      return carry

    lax.fori_loop(0, NROWCH, cpout, 0)

  return k(dst)


def _sc_spmm(table, src, dst, width):
  @functools.partial(
      pl.kernel,
      out_type=jax.ShapeDtypeStruct((NC, N, width), jnp.float32),
      mesh=_MESH,
      compiler_params=pltpu.CompilerParams(use_tc_tiling_on_sc=False),
      scratch_types=[
          pltpu.VMEM((CH,), jnp.int32),
          pltpu.VMEM((CH,), jnp.int32),
          pltpu.VMEM((CH, width), jnp.float32),
          pltpu.VMEM_SHARED((N, width), jnp.float32),
          pltpu.SemaphoreType.DMA,
      ],
  )
  def k(table_hbm, src_hbm, dst_hbm, out_hbm, sidx, didx, rows, acc, sem):
    c = lax.axis_index("c")
    s = lax.axis_index("s")
    gid = c * NS + s

    _fill(rows, CH, width, 0.0)

    def zacc(j, carry):
      @pl.when(lax.rem(j, NS) == s)
      def _():
        pltpu.sync_copy(rows, acc.at[pl.ds(j * ROWB, ROWB)])
      return carry

    lax.fori_loop(0, NROWCH, zacc, 0)
    plsc.subcore_barrier()

    base = gid * EPW

    def step(j, carry):
      off = pl.multiple_of(base + j * CH, 8)
      pltpu.sync_copy(src_hbm.at[pl.ds(off, CH)], sidx)
      pltpu.sync_copy(dst_hbm.at[pl.ds(off, CH)], didx)
      pltpu.async_copy(table_hbm.at[sidx], rows, sem).wait()
      pltpu.sync_copy(rows, acc.at[didx], add=True)
      return carry

    lax.fori_loop(0, NCHUNK, step, 0)
    plsc.subcore_barrier()

    def cpout(j, carry):
      @pl.when(lax.rem(j, NS) == s)
      def _():
        pltpu.sync_copy(acc.at[pl.ds(j * ROWB, ROWB)], rows.at[pl.ds(0, ROWB)])
        pltpu.sync_copy(rows.at[pl.ds(0, ROWB)],
                        out_hbm.at[c, pl.ds(j * ROWB, ROWB)])
      return carry

    lax.fori_loop(0, NROWCH, cpout, 0)

  return k(table, src, dst)


def _sc_edge_gather(u, z, src, dst):
  @functools.partial(
      pl.kernel,
      out_type=(jax.ShapeDtypeStruct((E, OUT_CH), jnp.float32),
                jax.ShapeDtypeStruct((E, OUT_CH), jnp.float32)),
      mesh=_MESH,
      compiler_params=pltpu.CompilerParams(use_tc_tiling_on_sc=False),
      scratch_types=[
          pltpu.VMEM((CH,), jnp.int32),
          pltpu.VMEM((CH,), jnp.int32),
          pltpu.VMEM((CH, OUT_CH), jnp.float32),
          pltpu.VMEM((CH, OUT_CH), jnp.float32),
          pltpu.SemaphoreType.DMA,
      ],
  )
  def k(u_hbm, z_hbm, src_hbm, dst_hbm, su_hbm, dz_hbm,
        sidx, didx, ubuf, zbuf, sem):
    c = lax.axis_index("c")
    s = lax.axis_index("s")
    gid = c * NS + s
    base = gid * EPW

    def step(j, carry):
      off = pl.multiple_of(base + j * CH, 8)
      pltpu.sync_copy(src_hbm.at[pl.ds(off, CH)], sidx)
      pltpu.sync_copy(dst_hbm.at[pl.ds(off, CH)], didx)
      cp_u = pltpu.async_copy(u_hbm.at[sidx], ubuf, sem)
      cp_z = pltpu.async_copy(z_hbm.at[didx], zbuf, sem)
      cp_u.wait()
      cp_z.wait()
      pltpu.sync_copy(ubuf, su_hbm.at[pl.ds(off, CH)])
      pltpu.sync_copy(zbuf, dz_hbm.at[pl.ds(off, CH)])
      return carry

    lax.fori_loop(0, NCHUNK, step, 0)

  return k(u, z, src, dst)


_NBLK = 1000
_GRID = N // _NBLK


def _dinv_from(dega_ref, degb_ref):
  deg = 1.0 + dega_ref[:, 0:1] + degb_ref[:, 0:1]
  return lax.rsqrt(deg)


def _tc_a(x, W1, dega, degb):
  def body(x_ref, w_ref, da_ref, db_ref, q_ref):
    dinv = _dinv_from(da_ref, db_ref)
    p = jnp.dot(x_ref[:, :], w_ref[:, :], preferred_element_type=jnp.float32)
    q_ref[:, :] = dinv * p

  return pl.pallas_call(
      body,
      grid=(_GRID,),
      in_specs=[
          pl.BlockSpec((_NBLK, IN_CH), lambda i: (i, 0)),
          pl.BlockSpec((IN_CH, HID), lambda i: (0, 0)),
          pl.BlockSpec((_NBLK, 16), lambda i: (i, 0)),
          pl.BlockSpec((_NBLK, 16), lambda i: (i, 0)),
      ],
      out_specs=pl.BlockSpec((_NBLK, HID), lambda i: (i, 0)),
      out_shape=jax.ShapeDtypeStruct((N, HID), jnp.float32),
  )(x, W1, dega, degb)


def _tc_c(s1a, s1b, q1, dega, degb, b1, W2):
  def body(a_ref, b_ref, q_ref, da_ref, db_ref, bias_ref, w_ref, out_ref):
    dinv = _dinv_from(da_ref, db_ref)
    h = dinv * (a_ref[:, :] + b_ref[:, :] + q_ref[:, :]) + bias_ref[:, :]
    h = jnp.maximum(h, 0.0)
    p2 = jnp.dot(h, w_ref[:, :], preferred_element_type=jnp.float32)
    out_ref[:, :] = dinv * p2

  return pl.pallas_call(
      body,
      grid=(_GRID,),
      in_specs=[
          pl.BlockSpec((_NBLK, HID), lambda i: (i, 0)),
          pl.BlockSpec((_NBLK, HID), lambda i: (i, 0)),
          pl.BlockSpec((_NBLK, HID), lambda i: (i, 0)),
          pl.BlockSpec((_NBLK, 16), lambda i: (i, 0)),
          pl.BlockSpec((_NBLK, 16), lambda i: (i, 0)),
          pl.BlockSpec((1, HID), lambda i: (0, 0)),
          pl.BlockSpec((HID, OUT_CH), lambda i: (0, 0)),
      ],
      out_specs=pl.BlockSpec((_NBLK, OUT_CH), lambda i: (i, 0)),
      out_shape=jax.ShapeDtypeStruct((N, OUT_CH), jnp.float32),
  )(s1a, s1b, q1, dega, degb, b1, W2)


def _tc_e(s2a, s2b, q2, dega, degb, b2, Wb0):
  def body(a_ref, b_ref, q_ref, da_ref, db_ref, bias_ref, w_ref,
           z_ref, u_ref):
    dinv = _dinv_from(da_ref, db_ref)
    z = dinv * (a_ref[:, :] + b_ref[:, :] + q_ref[:, :]) + bias_ref[:, :]
    z_ref[:, :] = z
    u_ref[:, :] = jnp.dot(z, w_ref[:, :], preferred_element_type=jnp.float32)

  return pl.pallas_call(
      body,
      grid=(_GRID,),
      in_specs=[
          pl.BlockSpec((_NBLK, OUT_CH), lambda i: (i, 0)),
          pl.BlockSpec((_NBLK, OUT_CH), lambda i: (i, 0)),
          pl.BlockSpec((_NBLK, OUT_CH), lambda i: (i, 0)),
          pl.BlockSpec((_NBLK, 16), lambda i: (i, 0)),
          pl.BlockSpec((_NBLK, 16), lambda i: (i, 0)),
          pl.BlockSpec((1, OUT_CH), lambda i: (0, 0)),
          pl.BlockSpec((OUT_CH, OUT_CH), lambda i: (0, 0)),
      ],
      out_specs=[
          pl.BlockSpec((_NBLK, OUT_CH), lambda i: (i, 0)),
          pl.BlockSpec((_NBLK, OUT_CH), lambda i: (i, 0)),
      ],
      out_shape=[
          jax.ShapeDtypeStruct((N, OUT_CH), jnp.float32),
          jax.ShapeDtypeStruct((N, OUT_CH), jnp.float32),
      ],
  )(s2a, s2b, q2, dega, degb, b2, Wb0)


_EBLK = 4000


def _tc_g(su, dz, bb):
  def body(s_ref, d_ref, bb_ref, out_ref):
    logit = jnp.sum(s_ref[:, :] * d_ref[:, :], axis=1, keepdims=True)
    out_ref[:, :] = jax.nn.sigmoid(logit + bb_ref[:, :])

  return pl.pallas_call(
      body,
      grid=(E // _EBLK,),
      in_specs=[
          pl.BlockSpec((_EBLK, OUT_CH), lambda i: (i, 0)),
          pl.BlockSpec((_EBLK, OUT_CH), lambda i: (i, 0)),
          pl.BlockSpec((1, 1), lambda i: (0, 0)),
      ],
      out_specs=pl.BlockSpec((_EBLK, 1), lambda i: (i, 0)),
      out_shape=jax.ShapeDtypeStruct((E, 1), jnp.float32),
  )(su, dz, bb)


def kernel(x, edge_index, W1, b1, W2, b2, Wb, bb):
  src = edge_index[0]
  dst = edge_index[1]

  deg2 = _sc_deg(dst)
  dega, degb = deg2[0], deg2[1]

  q1 = _tc_a(x, W1, dega, degb)
  s1 = _sc_spmm(q1, src, dst, HID)
  q2 = _tc_c(s1[0], s1[1], q1, dega, degb, b1.reshape(1, HID), W2)
  s2 = _sc_spmm(q2, src, dst, OUT_CH)
  z, u = _tc_e(s2[0], s2[1], q2, dega, degb, b2.reshape(1, OUT_CH), Wb[0])
  su, dz = _sc_edge_gather(u, z, src, dst)
  return _tc_g(su, dz, bb.reshape(1, 1))


# speedup vs baseline: 10.9442x; 1.5023x over previous
import functools

import jax
import jax.numpy as jnp
from jax import lax
from jax.experimental import pallas as pl
from jax.experimental.pallas import tpu as pltpu
from jax.experimental.pallas import tpu_sc as plsc

N = 10000
E = 320000
IN_CH = 128
HID = 128
OUT_CH = 64

NC = 2
NS = 16
NW = NC * NS
EPW = E // NW
CH = 80
NCHUNK = EPW // CH
ROWB = 80
NROWCH = N // ROWB

_MESH = plsc.VectorSubcoreMesh(
    core_axis_name="c", subcore_axis_name="s", num_cores=NC, num_subcores=NS)


def _fill(buf, rows, width, value):
  vec = jnp.full((16,), value, jnp.float32)

  def body(r, carry):
    for j in range(width // 16):
      buf[r, pl.ds(j * 16, 16)] = vec
    return carry

  lax.fori_loop(0, rows, body, 0)


_DEGW = 8


def _sc_deg(dstr):
  @functools.partial(
      pl.kernel,
      out_type=jax.ShapeDtypeStruct((NC, N, 16), jnp.float32),
      mesh=_MESH,
      compiler_params=pltpu.CompilerParams(use_tc_tiling_on_sc=False),
      scratch_types=[
          pltpu.VMEM((NCHUNK, CH), jnp.int32),
          pltpu.VMEM((ROWB, 16), jnp.float32),
          pltpu.VMEM((ROWB, 16), jnp.float32),
          pltpu.VMEM_SHARED((N, 16), jnp.float32),
          pltpu.SemaphoreType.DMA,
          pltpu.SemaphoreType.DMA,
      ],
  )
  def k(dst_hbm, out_hbm, idxd, zrows, ones, acc, psem, ssem):
    c = lax.axis_index("c")
    s = lax.axis_index("s")
    gid = c * NS + s

    cp_idx = pltpu.async_copy(dst_hbm.at[gid], idxd, psem)
    _fill(zrows, ROWB, 16, 0.0)
    _fill(ones, ROWB, 16, 1.0)

    def zacc(j, carry):
      @pl.when(lax.rem(j, NS) == s)
      def _():
        pltpu.sync_copy(zrows, acc.at[pl.ds(j * ROWB, ROWB)])
      return carry

    lax.fori_loop(0, NROWCH, zacc, 0)
    cp_idx.wait()
    plsc.subcore_barrier()

    def step(j, carry):
      pltpu.async_copy(ones, acc.at[idxd.at[j]], ssem, add=True)

      @pl.when(j >= _DEGW)
      def _():
        pltpu.make_async_copy(ones, acc.at[idxd.at[j]], ssem).wait()
      return carry

    lax.fori_loop(0, NCHUNK, step, 0)

    def drain(j, carry):
      pltpu.make_async_copy(ones, acc.at[idxd.at[0]], ssem).wait()
      return carry

    lax.fori_loop(0, _DEGW, drain, 0)
    plsc.subcore_barrier()

    def cpout(j, carry):
      @pl.when(lax.rem(j, NS) == s)
      def _():
        pltpu.sync_copy(acc.at[pl.ds(j * ROWB, ROWB)], zrows)
        pltpu.sync_copy(zrows, out_hbm.at[c, pl.ds(j * ROWB, ROWB)])
      return carry

    lax.fori_loop(0, NROWCH, cpout, 0)

  return k(dstr)


def _sc_spmm(table, srcr, dstr, width):
  @functools.partial(
      pl.kernel,
      out_type=jax.ShapeDtypeStruct((NC, N, width), jnp.float32),
      mesh=_MESH,
      compiler_params=pltpu.CompilerParams(use_tc_tiling_on_sc=False),
      scratch_types=[
          pltpu.VMEM((NCHUNK, CH), jnp.int32),
          pltpu.VMEM((NCHUNK, CH), jnp.int32),
          pltpu.VMEM((CH, width), jnp.float32),
          pltpu.VMEM((CH, width), jnp.float32),
          pltpu.VMEM_SHARED((N, width), jnp.float32),
          pltpu.SemaphoreType.DMA,
          pltpu.SemaphoreType.DMA,
          pltpu.SemaphoreType.DMA,
          pltpu.SemaphoreType.DMA,
          pltpu.SemaphoreType.DMA,
      ],
  )
  def k(table_hbm, src_hbm, dst_hbm, out_hbm,
        idxs, idxd, rows0, rows1, acc, psem, g0, g1, s0, s1):
    c = lax.axis_index("c")
    s = lax.axis_index("s")
    gid = c * NS + s

    cp_si = pltpu.async_copy(src_hbm.at[gid], idxs, psem)
    cp_di = pltpu.async_copy(dst_hbm.at[gid], idxd, psem)

    _fill(rows0, CH, width, 0.0)

    def zacc(j, carry):
      @pl.when(lax.rem(j, NS) == s)
      def _():
        pltpu.sync_copy(rows0, acc.at[pl.ds(j * ROWB, ROWB)])
      return carry

    lax.fori_loop(0, NROWCH, zacc, 0)
    cp_si.wait()
    cp_di.wait()
    pltpu.async_copy(table_hbm.at[idxs.at[0]], rows0, g0)
    plsc.subcore_barrier()

    def step(i, carry):
      j0 = 2 * i
      j1 = 2 * i + 1
      pltpu.make_async_copy(table_hbm.at[idxs.at[j0]], rows0, g0).wait()

      @pl.when(i > 0)
      def _():
        pltpu.make_async_copy(rows1, acc.at[idxd.at[j1]], s1).wait()

      pltpu.async_copy(table_hbm.at[idxs.at[j1]], rows1, g1)
      pltpu.async_copy(rows0, acc.at[idxd.at[j0]], s0, add=True)
      pltpu.make_async_copy(table_hbm.at[idxs.at[j1]], rows1, g1).wait()
      pltpu.make_async_copy(rows0, acc.at[idxd.at[j0]], s0).wait()
      pltpu.async_copy(table_hbm.at[idxs.at[j0 + 2]], rows0, g0)
      pltpu.async_copy(rows1, acc.at[idxd.at[j1]], s1, add=True)
      return carry

    lax.fori_loop(0, NCHUNK // 2, step, 0)

    last = NCHUNK - 1
    pltpu.make_async_copy(rows1, acc.at[idxd.at[last]], s1).wait()
    pltpu.make_async_copy(table_hbm.at[idxs.at[last]], rows0, g0).wait()
    pltpu.async_copy(rows0, acc.at[idxd.at[last]], s0, add=True)
    pltpu.make_async_copy(rows0, acc.at[idxd.at[last]], s0).wait()
    plsc.subcore_barrier()

    def cpout(j, carry):
      @pl.when(lax.rem(j, NS) == s)
      def _():
        pltpu.sync_copy(acc.at[pl.ds(j * ROWB, ROWB)], rows0)
        pltpu.sync_copy(rows0, out_hbm.at[c, pl.ds(j * ROWB, ROWB)])
      return carry

    lax.fori_loop(0, NROWCH, cpout, 0)

  return k(table, srcr, dstr)


def _sc_edge_gather(u, z, srcr, dstr):
  @functools.partial(
      pl.kernel,
      out_type=(jax.ShapeDtypeStruct((E, OUT_CH), jnp.float32),
                jax.ShapeDtypeStruct((E, OUT_CH), jnp.float32)),
      mesh=_MESH,
      compiler_params=pltpu.CompilerParams(use_tc_tiling_on_sc=False),
      scratch_types=[
          pltpu.VMEM((NCHUNK, CH), jnp.int32),
          pltpu.VMEM((NCHUNK, CH), jnp.int32),
          pltpu.VMEM((CH, OUT_CH), jnp.float32),
          pltpu.VMEM((CH, OUT_CH), jnp.float32),
          pltpu.VMEM((CH, OUT_CH), jnp.float32),
          pltpu.VMEM((CH, OUT_CH), jnp.float32),
      ] + [pltpu.SemaphoreType.DMA] * 9,
  )
  def k(u_hbm, z_hbm, src_hbm, dst_hbm, su_hbm, dz_hbm,
        idxs, idxd, ubuf0, zbuf0, ubuf1, zbuf1,
        psem, gu0, gz0, gu1, gz1, wu0, wz0, wu1, wz1):
    c = lax.axis_index("c")
    s = lax.axis_index("s")
    gid = c * NS + s
    base = gid * EPW

    cp_si = pltpu.async_copy(src_hbm.at[gid], idxs, psem)
    cp_di = pltpu.async_copy(dst_hbm.at[gid], idxd, psem)
    cp_si.wait()
    cp_di.wait()
    pltpu.async_copy(u_hbm.at[idxs.at[0]], ubuf0, gu0)
    pltpu.async_copy(z_hbm.at[idxd.at[0]], zbuf0, gz0)

    def step(i, carry):
      j0 = 2 * i
      j1 = 2 * i + 1
      off0 = pl.multiple_of(base + j0 * CH, 8)
      off1 = pl.multiple_of(base + j1 * CH, 8)
      pltpu.make_async_copy(u_hbm.at[idxs.at[j0]], ubuf0, gu0).wait()
      pltpu.make_async_copy(z_hbm.at[idxd.at[j0]], zbuf0, gz0).wait()

      @pl.when(i > 0)
      def _():
        pltpu.make_async_copy(ubuf1, su_hbm.at[pl.ds(off1, CH)], wu1).wait()
        pltpu.make_async_copy(zbuf1, dz_hbm.at[pl.ds(off1, CH)], wz1).wait()

      pltpu.async_copy(u_hbm.at[idxs.at[j1]], ubuf1, gu1)
      pltpu.async_copy(z_hbm.at[idxd.at[j1]], zbuf1, gz1)
      pltpu.async_copy(ubuf0, su_hbm.at[pl.ds(off0, CH)], wu0)
      pltpu.async_copy(zbuf0, dz_hbm.at[pl.ds(off0, CH)], wz0)
      pltpu.make_async_copy(u_hbm.at[idxs.at[j1]], ubuf1, gu1).wait()
      pltpu.make_async_copy(z_hbm.at[idxd.at[j1]], zbuf1, gz1).wait()
      pltpu.make_async_copy(ubuf0, su_hbm.at[pl.ds(off0, CH)], wu0).wait()
      pltpu.make_async_copy(zbuf0, dz_hbm.at[pl.ds(off0, CH)], wz0).wait()
      pltpu.async_copy(u_hbm.at[idxs.at[j0 + 2]], ubuf0, gu0)
      pltpu.async_copy(z_hbm.at[idxd.at[j0 + 2]], zbuf0, gz0)
      pltpu.async_copy(ubuf1, su_hbm.at[pl.ds(off1, CH)], wu1)
      pltpu.async_copy(zbuf1, dz_hbm.at[pl.ds(off1, CH)], wz1)
      return carry

    lax.fori_loop(0, NCHUNK // 2, step, 0)

    last = NCHUNK - 1
    offl = pl.multiple_of(base + last * CH, 8)
    pltpu.make_async_copy(ubuf1, su_hbm.at[pl.ds(offl, CH)], wu1).wait()
    pltpu.make_async_copy(zbuf1, dz_hbm.at[pl.ds(offl, CH)], wz1).wait()
    pltpu.make_async_copy(u_hbm.at[idxs.at[last]], ubuf0, gu0).wait()
    pltpu.make_async_copy(z_hbm.at[idxd.at[last]], zbuf0, gz0).wait()
    pltpu.sync_copy(ubuf0, su_hbm.at[pl.ds(offl, CH)])
    pltpu.sync_copy(zbuf0, dz_hbm.at[pl.ds(offl, CH)])

  return k(u, z, srcr, dstr)


_NBLK = 1000
_GRID = N // _NBLK


def _dinv_from(dega_ref, degb_ref):
  deg = 1.0 + dega_ref[:, 0:1] + degb_ref[:, 0:1]
  return lax.rsqrt(deg)


def _tc_a(x, W1, dega, degb):
  def body(x_ref, w_ref, da_ref, db_ref, q_ref):
    dinv = _dinv_from(da_ref, db_ref)
    p = jnp.dot(x_ref[:, :], w_ref[:, :], preferred_element_type=jnp.float32)
    q_ref[:, :] = dinv * p

  return pl.pallas_call(
      body,
      grid=(_GRID,),
      in_specs=[
          pl.BlockSpec((_NBLK, IN_CH), lambda i: (i, 0)),
          pl.BlockSpec((IN_CH, HID), lambda i: (0, 0)),
          pl.BlockSpec((_NBLK, 16), lambda i: (i, 0)),
          pl.BlockSpec((_NBLK, 16), lambda i: (i, 0)),
      ],
      out_specs=pl.BlockSpec((_NBLK, HID), lambda i: (i, 0)),
      out_shape=jax.ShapeDtypeStruct((N, HID), jnp.float32),
  )(x, W1, dega, degb)


def _tc_c(s1a, s1b, q1, dega, degb, b1, W2):
  def body(a_ref, b_ref, q_ref, da_ref, db_ref, bias_ref, w_ref, out_ref):
    dinv = _dinv_from(da_ref, db_ref)
    h = dinv * (a_ref[:, :] + b_ref[:, :] + q_ref[:, :]) + bias_ref[:, :]
    h = jnp.maximum(h, 0.0)
    p2 = jnp.dot(h, w_ref[:, :], preferred_element_type=jnp.float32)
    out_ref[:, :] = dinv * p2

  return pl.pallas_call(
      body,
      grid=(_GRID,),
      in_specs=[
          pl.BlockSpec((_NBLK, HID), lambda i: (i, 0)),
          pl.BlockSpec((_NBLK, HID), lambda i: (i, 0)),
          pl.BlockSpec((_NBLK, HID), lambda i: (i, 0)),
          pl.BlockSpec((_NBLK, 16), lambda i: (i, 0)),
          pl.BlockSpec((_NBLK, 16), lambda i: (i, 0)),
          pl.BlockSpec((1, HID), lambda i: (0, 0)),
          pl.BlockSpec((HID, OUT_CH), lambda i: (0, 0)),
      ],
      out_specs=pl.BlockSpec((_NBLK, OUT_CH), lambda i: (i, 0)),
      out_shape=jax.ShapeDtypeStruct((N, OUT_CH), jnp.float32),
  )(s1a, s1b, q1, dega, degb, b1, W2)


def _tc_e(s2a, s2b, q2, dega, degb, b2, Wb0):
  def body(a_ref, b_ref, q_ref, da_ref, db_ref, bias_ref, w_ref,
           z_ref, u_ref):
    dinv = _dinv_from(da_ref, db_ref)
    z = dinv * (a_ref[:, :] + b_ref[:, :] + q_ref[:, :]) + bias_ref[:, :]
    z_ref[:, :] = z
    u_ref[:, :] = jnp.dot(z, w_ref[:, :], preferred_element_type=jnp.float32)

  return pl.pallas_call(
      body,
      grid=(_GRID,),
      in_specs=[
          pl.BlockSpec((_NBLK, OUT_CH), lambda i: (i, 0)),
          pl.BlockSpec((_NBLK, OUT_CH), lambda i: (i, 0)),
          pl.BlockSpec((_NBLK, OUT_CH), lambda i: (i, 0)),
          pl.BlockSpec((_NBLK, 16), lambda i: (i, 0)),
          pl.BlockSpec((_NBLK, 16), lambda i: (i, 0)),
          pl.BlockSpec((1, OUT_CH), lambda i: (0, 0)),
          pl.BlockSpec((OUT_CH, OUT_CH), lambda i: (0, 0)),
      ],
      out_specs=[
          pl.BlockSpec((_NBLK, OUT_CH), lambda i: (i, 0)),
          pl.BlockSpec((_NBLK, OUT_CH), lambda i: (i, 0)),
      ],
      out_shape=[
          jax.ShapeDtypeStruct((N, OUT_CH), jnp.float32),
          jax.ShapeDtypeStruct((N, OUT_CH), jnp.float32),
      ],
  )(s2a, s2b, q2, dega, degb, b2, Wb0)


_EBLK = 4000


def _tc_g(su, dz, bb):
  def body(s_ref, d_ref, bb_ref, out_ref):
    logit = jnp.sum(s_ref[:, :] * d_ref[:, :], axis=1, keepdims=True)
    out_ref[:, :] = jax.nn.sigmoid(logit + bb_ref[:, :])

  return pl.pallas_call(
      body,
      grid=(E // _EBLK,),
      in_specs=[
          pl.BlockSpec((_EBLK, OUT_CH), lambda i: (i, 0)),
          pl.BlockSpec((_EBLK, OUT_CH), lambda i: (i, 0)),
          pl.BlockSpec((1, 1), lambda i: (0, 0)),
      ],
      out_specs=pl.BlockSpec((_EBLK, 1), lambda i: (i, 0)),
      out_shape=jax.ShapeDtypeStruct((E, 1), jnp.float32),
  )(su, dz, bb)


def kernel(x, edge_index, W1, b1, W2, b2, Wb, bb):
  srcr = edge_index[0].reshape(NW, NCHUNK, CH)
  dstr = edge_index[1].reshape(NW, NCHUNK, CH)

  deg2 = _sc_deg(dstr)
  dega, degb = deg2[0], deg2[1]

  q1 = _tc_a(x, W1, dega, degb)
  s1 = _sc_spmm(q1, srcr, dstr, HID)
  q2 = _tc_c(s1[0], s1[1], q1, dega, degb, b1.reshape(1, HID), W2)
  s2 = _sc_spmm(q2, srcr, dstr, OUT_CH)
  z, u = _tc_e(s2[0], s2[1], q2, dega, degb, b2.reshape(1, OUT_CH), Wb[0])
  su, dz = _sc_edge_gather(u, z, srcr, dstr)
  return _tc_g(su, dz, bb.reshape(1, 1))


# speedup vs baseline: 21.3871x; 1.9542x over previous
import functools

import jax
import jax.numpy as jnp
from jax import lax
from jax.experimental import pallas as pl
from jax.experimental.pallas import tpu as pltpu
from jax.experimental.pallas import tpu_sc as plsc

N = 10000
E = 320000
IN_CH = 128
HID = 128
OUT_CH = 64

NC = 2
NS = 16
NW = NC * NS
EPW = E // NW
CH = 80
NCHUNK = EPW // CH
ROWB = 80
NROWCH = N // ROWB

_MESH = plsc.VectorSubcoreMesh(
    core_axis_name="c", subcore_axis_name="s", num_cores=NC, num_subcores=NS)


def _fill(buf, rows, width, value):
  vec = jnp.full((16,), value, jnp.float32)

  def body(r, carry):
    for j in range(width // 16):
      buf[r, pl.ds(j * 16, 16)] = vec
    return carry

  lax.fori_loop(0, rows, body, 0)


_DEGW = 8


def _sc_deg(dstr):
  @functools.partial(
      pl.kernel,
      out_type=jax.ShapeDtypeStruct((NC, N, 16), jnp.float32),
      mesh=_MESH,
      compiler_params=pltpu.CompilerParams(use_tc_tiling_on_sc=False),
      scratch_types=[
          pltpu.VMEM((NCHUNK, CH), jnp.int32),
          pltpu.VMEM((ROWB, 16), jnp.float32),
          pltpu.VMEM((ROWB, 16), jnp.float32),
          pltpu.VMEM_SHARED((N, 16), jnp.float32),
          pltpu.SemaphoreType.DMA,
          pltpu.SemaphoreType.DMA,
      ],
  )
  def k(dst_hbm, out_hbm, idxd, zrows, ones, acc, psem, ssem):
    c = lax.axis_index("c")
    s = lax.axis_index("s")
    gid = c * NS + s

    cp_idx = pltpu.async_copy(dst_hbm.at[gid], idxd, psem)
    _fill(zrows, ROWB, 16, 0.0)
    _fill(ones, ROWB, 16, 1.0)

    def zacc(j, carry):
      @pl.when(lax.rem(j, NS) == s)
      def _():
        pltpu.sync_copy(zrows, acc.at[pl.ds(j * ROWB, ROWB)])
      return carry

    lax.fori_loop(0, NROWCH, zacc, 0)
    cp_idx.wait()
    plsc.subcore_barrier()

    def step(j, carry):
      pltpu.async_copy(ones, acc.at[idxd.at[j]], ssem, add=True)

      @pl.when(j >= _DEGW)
      def _():
        pltpu.make_async_copy(ones, acc.at[idxd.at[j]], ssem).wait()
      return carry

    lax.fori_loop(0, NCHUNK, step, 0)

    def drain(j, carry):
      pltpu.make_async_copy(ones, acc.at[idxd.at[0]], ssem).wait()
      return carry

    lax.fori_loop(0, _DEGW, drain, 0)
    plsc.subcore_barrier()

    def cpout(j, carry):
      @pl.when(lax.rem(j, NS) == s)
      def _():
        pltpu.sync_copy(acc.at[pl.ds(j * ROWB, ROWB)], zrows)
        pltpu.sync_copy(zrows, out_hbm.at[c, pl.ds(j * ROWB, ROWB)])
      return carry

    lax.fori_loop(0, NROWCH, cpout, 0)

  return k(dstr)


def _sc_spmm(table, srcr, dstr, width):
  @functools.partial(
      pl.kernel,
      out_type=jax.ShapeDtypeStruct((NC, N, width), jnp.float32),
      mesh=_MESH,
      compiler_params=pltpu.CompilerParams(use_tc_tiling_on_sc=False),
      scratch_types=[
          pltpu.VMEM((NCHUNK, CH), jnp.int32),
          pltpu.VMEM((NCHUNK, CH), jnp.int32),
          pltpu.VMEM((CH, width), jnp.float32),
          pltpu.VMEM((CH, width), jnp.float32),
          pltpu.VMEM_SHARED((N, width), jnp.float32),
          pltpu.SemaphoreType.DMA,
          pltpu.SemaphoreType.DMA,
          pltpu.SemaphoreType.DMA,
          pltpu.SemaphoreType.DMA,
          pltpu.SemaphoreType.DMA,
      ],
  )
  def k(table_hbm, src_hbm, dst_hbm, out_hbm,
        idxs, idxd, rows0, rows1, acc, psem, g0, g1, s0, s1):
    c = lax.axis_index("c")
    s = lax.axis_index("s")
    gid = c * NS + s

    cp_si = pltpu.async_copy(src_hbm.at[gid], idxs, psem)
    cp_di = pltpu.async_copy(dst_hbm.at[gid], idxd, psem)

    _fill(rows0, CH, width, 0.0)

    def zacc(j, carry):
      @pl.when(lax.rem(j, NS) == s)
      def _():
        pltpu.sync_copy(rows0, acc.at[pl.ds(j * ROWB, ROWB)])
      return carry

    lax.fori_loop(0, NROWCH, zacc, 0)
    cp_si.wait()
    cp_di.wait()
    pltpu.async_copy(table_hbm.at[idxs.at[0]], rows0, g0)
    plsc.subcore_barrier()

    def step(i, carry):
      j0 = 2 * i
      j1 = 2 * i + 1
      pltpu.make_async_copy(table_hbm.at[idxs.at[j0]], rows0, g0).wait()

      @pl.when(i > 0)
      def _():
        pltpu.make_async_copy(rows1, acc.at[idxd.at[j1]], s1).wait()

      pltpu.async_copy(table_hbm.at[idxs.at[j1]], rows1, g1)
      pltpu.async_copy(rows0, acc.at[idxd.at[j0]], s0, add=True)
      pltpu.make_async_copy(table_hbm.at[idxs.at[j1]], rows1, g1).wait()
      pltpu.make_async_copy(rows0, acc.at[idxd.at[j0]], s0).wait()
      pltpu.async_copy(table_hbm.at[idxs.at[j0 + 2]], rows0, g0)
      pltpu.async_copy(rows1, acc.at[idxd.at[j1]], s1, add=True)
      return carry

    lax.fori_loop(0, NCHUNK // 2, step, 0)

    last = NCHUNK - 1
    pltpu.make_async_copy(rows1, acc.at[idxd.at[last]], s1).wait()
    pltpu.make_async_copy(table_hbm.at[idxs.at[last]], rows0, g0).wait()
    pltpu.async_copy(rows0, acc.at[idxd.at[last]], s0, add=True)
    pltpu.make_async_copy(rows0, acc.at[idxd.at[last]], s0).wait()
    plsc.subcore_barrier()

    def cpout(j, carry):
      @pl.when(lax.rem(j, NS) == s)
      def _():
        pltpu.sync_copy(acc.at[pl.ds(j * ROWB, ROWB)], rows0)
        pltpu.sync_copy(rows0, out_hbm.at[c, pl.ds(j * ROWB, ROWB)])
      return carry

    lax.fori_loop(0, NROWCH, cpout, 0)

  return k(table, srcr, dstr)


_NQ = OUT_CH // 16


def _sc_edge_decode(u, z, srcr, dstr, bb16):
  @functools.partial(
      pl.kernel,
      out_type=jax.ShapeDtypeStruct((E,), jnp.float32),
      mesh=_MESH,
      compiler_params=pltpu.CompilerParams(
          use_tc_tiling_on_sc=False, needs_layout_passes=False),
      scratch_types=[
          pltpu.VMEM((NCHUNK, CH), jnp.int32),
          pltpu.VMEM((NCHUNK, CH), jnp.int32),
          pltpu.VMEM((CH, OUT_CH), jnp.float32),
          pltpu.VMEM((CH, OUT_CH), jnp.float32),
          pltpu.VMEM((CH, OUT_CH), jnp.float32),
          pltpu.VMEM((CH, OUT_CH), jnp.float32),
          pltpu.VMEM((CH,), jnp.float32),
          pltpu.VMEM((CH,), jnp.float32),
          pltpu.VMEM((16,), jnp.float32),
      ] + [pltpu.SemaphoreType.DMA] * 7,
  )
  def k(u_hbm, z_hbm, src_hbm, dst_hbm, bb_hbm, out_hbm,
        idxs, idxd, ubuf0, zbuf0, ubuf1, zbuf1, obuf0, obuf1, bbv,
        psem, gu0, gz0, gu1, gz1, wo0, wo1):
    c = lax.axis_index("c")
    s = lax.axis_index("s")
    gid = c * NS + s
    base = gid * EPW

    cp_si = pltpu.async_copy(src_hbm.at[gid], idxs, psem)
    cp_di = pltpu.async_copy(dst_hbm.at[gid], idxd, psem)
    pltpu.sync_copy(bb_hbm, bbv)
    bias = bbv[...]
    lane = lax.iota(jnp.int32, 16)
    cp_si.wait()
    cp_di.wait()
    pltpu.async_copy(u_hbm.at[idxs.at[0]], ubuf0, gu0)
    pltpu.async_copy(z_hbm.at[idxd.at[0]], zbuf0, gz0)

    def dot_chunk(ubuf, zbuf, obuf):
      def grp(g, carry):
        res = jnp.zeros((16,), jnp.float32)
        for e in range(16):
          row = g * 16 + e
          acc = ubuf[row, pl.ds(0, 16)] * zbuf[row, pl.ds(0, 16)]
          for q in range(1, _NQ):
            acc = acc + ubuf[row, pl.ds(q * 16, 16)] * zbuf[row, pl.ds(q * 16, 16)]
          res = jnp.where(lane == e, jnp.full((16,), jnp.sum(acc)), res)
        obuf[pl.ds(g * 16, 16)] = 1.0 / (1.0 + jnp.exp(-(res + bias)))
        return carry

      lax.fori_loop(0, CH // 16, grp, 0)

    def step(i, carry):
      j0 = 2 * i
      j1 = 2 * i + 1
      off0 = pl.multiple_of(base + j0 * CH, 8)
      off1 = pl.multiple_of(base + j1 * CH, 8)
      pltpu.make_async_copy(u_hbm.at[idxs.at[j0]], ubuf0, gu0).wait()
      pltpu.make_async_copy(z_hbm.at[idxd.at[j0]], zbuf0, gz0).wait()
      pltpu.async_copy(u_hbm.at[idxs.at[j1]], ubuf1, gu1)
      pltpu.async_copy(z_hbm.at[idxd.at[j1]], zbuf1, gz1)

      @pl.when(i > 0)
      def _():
        pltpu.make_async_copy(obuf0, out_hbm.at[pl.ds(off0, CH)], wo0).wait()

      dot_chunk(ubuf0, zbuf0, obuf0)
      pltpu.async_copy(obuf0, out_hbm.at[pl.ds(off0, CH)], wo0)
      pltpu.make_async_copy(u_hbm.at[idxs.at[j1]], ubuf1, gu1).wait()
      pltpu.make_async_copy(z_hbm.at[idxd.at[j1]], zbuf1, gz1).wait()
      pltpu.async_copy(u_hbm.at[idxs.at[j0 + 2]], ubuf0, gu0)
      pltpu.async_copy(z_hbm.at[idxd.at[j0 + 2]], zbuf0, gz0)

      @pl.when(i > 0)
      def _():
        pltpu.make_async_copy(obuf1, out_hbm.at[pl.ds(off1, CH)], wo1).wait()

      dot_chunk(ubuf1, zbuf1, obuf1)
      pltpu.async_copy(obuf1, out_hbm.at[pl.ds(off1, CH)], wo1)
      return carry

    lax.fori_loop(0, NCHUNK // 2, step, 0)

    last = NCHUNK - 1
    offl = pl.multiple_of(base + last * CH, 8)
    pltpu.make_async_copy(u_hbm.at[idxs.at[last]], ubuf0, gu0).wait()
    pltpu.make_async_copy(z_hbm.at[idxd.at[last]], zbuf0, gz0).wait()
    pltpu.make_async_copy(obuf0, out_hbm.at[pl.ds(offl, CH)], wo0).wait()
    dot_chunk(ubuf0, zbuf0, obuf0)
    pltpu.sync_copy(obuf0, out_hbm.at[pl.ds(offl, CH)])
    pltpu.make_async_copy(obuf1, out_hbm.at[pl.ds(offl, CH)], wo1).wait()

  return k(u, z, srcr, dstr, bb16)


_NBLK = 1000
_GRID = N // _NBLK


def _dinv_from(dega_ref, degb_ref):
  deg = 1.0 + dega_ref[:, 0:1] + degb_ref[:, 0:1]
  return lax.rsqrt(deg)


def _tc_a(x, W1, dega, degb):
  def body(x_ref, w_ref, da_ref, db_ref, q_ref):
    dinv = _dinv_from(da_ref, db_ref)
    p = jnp.dot(x_ref[:, :], w_ref[:, :], preferred_element_type=jnp.float32)
    q_ref[:, :] = dinv * p

  return pl.pallas_call(
      body,
      grid=(_GRID,),
      in_specs=[
          pl.BlockSpec((_NBLK, IN_CH), lambda i: (i, 0)),
          pl.BlockSpec((IN_CH, HID), lambda i: (0, 0)),
          pl.BlockSpec((_NBLK, 16), lambda i: (i, 0)),
          pl.BlockSpec((_NBLK, 16), lambda i: (i, 0)),
      ],
      out_specs=pl.BlockSpec((_NBLK, HID), lambda i: (i, 0)),
      out_shape=jax.ShapeDtypeStruct((N, HID), jnp.float32),
  )(x, W1, dega, degb)


def _tc_c(s1a, s1b, q1, dega, degb, b1, W2):
  def body(a_ref, b_ref, q_ref, da_ref, db_ref, bias_ref, w_ref, out_ref):
    dinv = _dinv_from(da_ref, db_ref)
    h = dinv * (a_ref[:, :] + b_ref[:, :] + q_ref[:, :]) + bias_ref[:, :]
    h = jnp.maximum(h, 0.0)
    p2 = jnp.dot(h, w_ref[:, :], preferred_element_type=jnp.float32)
    out_ref[:, :] = dinv * p2

  return pl.pallas_call(
      body,
      grid=(_GRID,),
      in_specs=[
          pl.BlockSpec((_NBLK, HID), lambda i: (i, 0)),
          pl.BlockSpec((_NBLK, HID), lambda i: (i, 0)),
          pl.BlockSpec((_NBLK, HID), lambda i: (i, 0)),
          pl.BlockSpec((_NBLK, 16), lambda i: (i, 0)),
          pl.BlockSpec((_NBLK, 16), lambda i: (i, 0)),
          pl.BlockSpec((1, HID), lambda i: (0, 0)),
          pl.BlockSpec((HID, OUT_CH), lambda i: (0, 0)),
      ],
      out_specs=pl.BlockSpec((_NBLK, OUT_CH), lambda i: (i, 0)),
      out_shape=jax.ShapeDtypeStruct((N, OUT_CH), jnp.float32),
  )(s1a, s1b, q1, dega, degb, b1, W2)


def _tc_e(s2a, s2b, q2, dega, degb, b2, Wb0):
  def body(a_ref, b_ref, q_ref, da_ref, db_ref, bias_ref, w_ref,
           z_ref, u_ref):
    dinv = _dinv_from(da_ref, db_ref)
    z = dinv * (a_ref[:, :] + b_ref[:, :] + q_ref[:, :]) + bias_ref[:, :]
    z_ref[:, :] = z
    u_ref[:, :] = jnp.dot(z, w_ref[:, :], preferred_element_type=jnp.float32)

  return pl.pallas_call(
      body,
      grid=(_GRID,),
      in_specs=[
          pl.BlockSpec((_NBLK, OUT_CH), lambda i: (i, 0)),
          pl.BlockSpec((_NBLK, OUT_CH), lambda i: (i, 0)),
          pl.BlockSpec((_NBLK, OUT_CH), lambda i: (i, 0)),
          pl.BlockSpec((_NBLK, 16), lambda i: (i, 0)),
          pl.BlockSpec((_NBLK, 16), lambda i: (i, 0)),
          pl.BlockSpec((1, OUT_CH), lambda i: (0, 0)),
          pl.BlockSpec((OUT_CH, OUT_CH), lambda i: (0, 0)),
      ],
      out_specs=[
          pl.BlockSpec((_NBLK, OUT_CH), lambda i: (i, 0)),
          pl.BlockSpec((_NBLK, OUT_CH), lambda i: (i, 0)),
      ],
      out_shape=[
          jax.ShapeDtypeStruct((N, OUT_CH), jnp.float32),
          jax.ShapeDtypeStruct((N, OUT_CH), jnp.float32),
      ],
  )(s2a, s2b, q2, dega, degb, b2, Wb0)


def kernel(x, edge_index, W1, b1, W2, b2, Wb, bb):
  srcr = edge_index[0].reshape(NW, NCHUNK, CH)
  dstr = edge_index[1].reshape(NW, NCHUNK, CH)

  deg2 = _sc_deg(dstr)
  dega, degb = deg2[0], deg2[1]

  q1 = _tc_a(x, W1, dega, degb)
  s1 = _sc_spmm(q1, srcr, dstr, HID)
  q2 = _tc_c(s1[0], s1[1], q1, dega, degb, b1.reshape(1, HID), W2)
  s2 = _sc_spmm(q2, srcr, dstr, OUT_CH)
  z, u = _tc_e(s2[0], s2[1], q2, dega, degb, b2.reshape(1, OUT_CH), Wb[0])
  bb16 = jnp.broadcast_to(bb.reshape(1), (16,))
  return _sc_edge_decode(u, z, srcr, dstr, bb16).reshape(E, 1)


# speedup vs baseline: 21.4691x; 1.0038x over previous
import functools

import jax
import jax.numpy as jnp
from jax import lax
from jax.experimental import pallas as pl
from jax.experimental.pallas import tpu as pltpu
from jax.experimental.pallas import tpu_sc as plsc

N = 10000
E = 320000
IN_CH = 128
HID = 128
OUT_CH = 64

NC = 2
NS = 16
NW = NC * NS
EPW = E // NW
CH = 80
NCHUNK = EPW // CH
ROWB = 80
NROWCH = N // ROWB

_MESH = plsc.VectorSubcoreMesh(
    core_axis_name="c", subcore_axis_name="s", num_cores=NC, num_subcores=NS)


def _fill(buf, rows, width, value):
  vec = jnp.full((16,), value, jnp.float32)

  def body(r, carry):
    for j in range(width // 16):
      buf[r, pl.ds(j * 16, 16)] = vec
    return carry

  lax.fori_loop(0, rows, body, 0)


_DEGW = 8


def _sc_deg(dstr):
  @functools.partial(
      pl.kernel,
      out_type=jax.ShapeDtypeStruct((NC, N, 16), jnp.float32),
      mesh=_MESH,
      compiler_params=pltpu.CompilerParams(use_tc_tiling_on_sc=False),
      scratch_types=[
          pltpu.VMEM((NCHUNK, CH), jnp.int32),
          pltpu.VMEM((ROWB, 16), jnp.float32),
          pltpu.VMEM((ROWB, 16), jnp.float32),
          pltpu.VMEM_SHARED((N, 16), jnp.float32),
          pltpu.SemaphoreType.DMA,
          pltpu.SemaphoreType.DMA,
      ],
  )
  def k(dst_hbm, out_hbm, idxd, zrows, ones, acc, psem, ssem):
    c = lax.axis_index("c")
    s = lax.axis_index("s")
    gid = c * NS + s

    cp_idx = pltpu.async_copy(dst_hbm.at[gid], idxd, psem)
    _fill(zrows, ROWB, 16, 0.0)
    _fill(ones, ROWB, 16, 1.0)

    def zacc(j, carry):
      @pl.when(lax.rem(j, NS) == s)
      def _():
        pltpu.sync_copy(zrows, acc.at[pl.ds(j * ROWB, ROWB)])
      return carry

    lax.fori_loop(0, NROWCH, zacc, 0)
    cp_idx.wait()
    plsc.subcore_barrier()

    def step(j, carry):
      pltpu.async_copy(ones, acc.at[idxd.at[j]], ssem, add=True)

      @pl.when(j >= _DEGW)
      def _():
        pltpu.make_async_copy(ones, acc.at[idxd.at[j]], ssem).wait()
      return carry

    lax.fori_loop(0, NCHUNK, step, 0)

    def drain(j, carry):
      pltpu.make_async_copy(ones, acc.at[idxd.at[0]], ssem).wait()
      return carry

    lax.fori_loop(0, _DEGW, drain, 0)
    plsc.subcore_barrier()

    def cpout(j, carry):
      @pl.when(lax.rem(j, NS) == s)
      def _():
        pltpu.sync_copy(acc.at[pl.ds(j * ROWB, ROWB)], zrows)
        pltpu.sync_copy(zrows, out_hbm.at[c, pl.ds(j * ROWB, ROWB)])
      return carry

    lax.fori_loop(0, NROWCH, cpout, 0)

  return k(dstr)


_NB = 2


def _sc_spmm(table, srcr, dstr, width):
  @functools.partial(
      pl.kernel,
      out_type=jax.ShapeDtypeStruct((NC, N, width), jnp.float32),
      mesh=_MESH,
      compiler_params=pltpu.CompilerParams(use_tc_tiling_on_sc=False),
      scratch_types=[
          pltpu.VMEM((NCHUNK, CH), jnp.int32),
          pltpu.VMEM((NCHUNK, CH), jnp.int32),
      ] + [pltpu.VMEM((CH, width), jnp.float32)] * _NB + [
          pltpu.VMEM_SHARED((N, width), jnp.float32),
          pltpu.SemaphoreType.DMA,
      ] + [pltpu.SemaphoreType.DMA] * (2 * _NB),
  )
  def k(table_hbm, src_hbm, dst_hbm, out_hbm, idxs, idxd, *bufs):
    rows = bufs[:_NB]
    acc = bufs[_NB]
    psem = bufs[_NB + 1]
    g = bufs[_NB + 2:_NB + 2 + _NB]
    st = bufs[_NB + 2 + _NB:]
    c = lax.axis_index("c")
    s = lax.axis_index("s")
    gid = c * NS + s

    cp_si = pltpu.async_copy(src_hbm.at[gid], idxs, psem)
    cp_di = pltpu.async_copy(dst_hbm.at[gid], idxd, psem)

    _fill(rows[0], CH, width, 0.0)

    def zacc(j, carry):
      @pl.when(lax.rem(j, NS) == s)
      def _():
        pltpu.sync_copy(rows[0], acc.at[pl.ds(j * ROWB, ROWB)])
      return carry

    lax.fori_loop(0, NROWCH, zacc, 0)
    cp_si.wait()
    cp_di.wait()
    for b in range(_NB - 1):
      pltpu.async_copy(table_hbm.at[idxs.at[b]], rows[b], g[b])
    plsc.subcore_barrier()

    def step(i, carry):
      for b in range(_NB):
        j = _NB * i + b
        pltpu.make_async_copy(table_hbm.at[idxs.at[j]], rows[b], g[b]).wait()
        bn = (b + _NB - 1) % _NB
        if b == 0:
          @pl.when(i > 0)
          def _():
            pltpu.make_async_copy(rows[bn], acc.at[idxd.at[j]], st[bn]).wait()
          pltpu.async_copy(table_hbm.at[idxs.at[j + _NB - 1]], rows[bn], g[bn])
        else:
          pltpu.make_async_copy(rows[bn], acc.at[idxd.at[j]], st[bn]).wait()

          @pl.when(i < NCHUNK // _NB - 1)
          def _():
            pltpu.async_copy(table_hbm.at[idxs.at[j + _NB - 1]], rows[bn], g[bn])
        pltpu.async_copy(rows[b], acc.at[idxd.at[j]], st[b], add=True)
      return carry

    lax.fori_loop(0, NCHUNK // _NB, step, 0)
    pltpu.make_async_copy(rows[_NB - 1], acc.at[idxd.at[0]],
                          st[_NB - 1]).wait()
    plsc.subcore_barrier()

    def cpout(j, carry):
      @pl.when(lax.rem(j, NS) == s)
      def _():
        pltpu.sync_copy(acc.at[pl.ds(j * ROWB, ROWB)], rows[0])
        pltpu.sync_copy(rows[0], out_hbm.at[c, pl.ds(j * ROWB, ROWB)])
      return carry

    lax.fori_loop(0, NROWCH, cpout, 0)

  return k(table, srcr, dstr)


_NQ = OUT_CH // 16


def _sc_edge_decode(u, z, srcr, dstr, bb16):
  @functools.partial(
      pl.kernel,
      out_type=jax.ShapeDtypeStruct((E,), jnp.float32),
      mesh=_MESH,
      compiler_params=pltpu.CompilerParams(
          use_tc_tiling_on_sc=False, needs_layout_passes=False),
      scratch_types=[
          pltpu.VMEM((NCHUNK, CH), jnp.int32),
          pltpu.VMEM((NCHUNK, CH), jnp.int32),
          pltpu.VMEM((CH, OUT_CH), jnp.float32),
          pltpu.VMEM((CH, OUT_CH), jnp.float32),
          pltpu.VMEM((CH, OUT_CH), jnp.float32),
          pltpu.VMEM((CH, OUT_CH), jnp.float32),
          pltpu.VMEM((CH,), jnp.float32),
          pltpu.VMEM((CH,), jnp.float32),
          pltpu.VMEM((16,), jnp.float32),
      ] + [pltpu.SemaphoreType.DMA] * 7,
  )
  def k(u_hbm, z_hbm, src_hbm, dst_hbm, bb_hbm, out_hbm,
        idxs, idxd, ubuf0, zbuf0, ubuf1, zbuf1, obuf0, obuf1, bbv,
        psem, gu0, gz0, gu1, gz1, wo0, wo1):
    c = lax.axis_index("c")
    s = lax.axis_index("s")
    gid = c * NS + s
    base = gid * EPW

    cp_si = pltpu.async_copy(src_hbm.at[gid], idxs, psem)
    cp_di = pltpu.async_copy(dst_hbm.at[gid], idxd, psem)
    pltpu.sync_copy(bb_hbm, bbv)
    bias = bbv[...]
    lane = lax.iota(jnp.int32, 16)
    cp_si.wait()
    cp_di.wait()
    pltpu.async_copy(u_hbm.at[idxs.at[0]], ubuf0, gu0)
    pltpu.async_copy(z_hbm.at[idxd.at[0]], zbuf0, gz0)

    def dot_chunk(ubuf, zbuf, obuf):
      def grp(g, carry):
        res = jnp.zeros((16,), jnp.float32)
        for e in range(16):
          row = g * 16 + e
          acc = ubuf[row, pl.ds(0, 16)] * zbuf[row, pl.ds(0, 16)]
          for q in range(1, _NQ):
            acc = acc + ubuf[row, pl.ds(q * 16, 16)] * zbuf[row, pl.ds(q * 16, 16)]
          res = jnp.where(lane == e, jnp.full((16,), jnp.sum(acc)), res)
        obuf[pl.ds(g * 16, 16)] = 1.0 / (1.0 + jnp.exp(-(res + bias)))
        return carry

      lax.fori_loop(0, CH // 16, grp, 0)

    def step(i, carry):
      j0 = 2 * i
      j1 = 2 * i + 1
      off0 = pl.multiple_of(base + j0 * CH, 8)
      off1 = pl.multiple_of(base + j1 * CH, 8)
      pltpu.make_async_copy(u_hbm.at[idxs.at[j0]], ubuf0, gu0).wait()
      pltpu.make_async_copy(z_hbm.at[idxd.at[j0]], zbuf0, gz0).wait()
      pltpu.async_copy(u_hbm.at[idxs.at[j1]], ubuf1, gu1)
      pltpu.async_copy(z_hbm.at[idxd.at[j1]], zbuf1, gz1)

      @pl.when(i > 0)
      def _():
        pltpu.make_async_copy(obuf0, out_hbm.at[pl.ds(off0, CH)], wo0).wait()

      dot_chunk(ubuf0, zbuf0, obuf0)
      pltpu.async_copy(obuf0, out_hbm.at[pl.ds(off0, CH)], wo0)
      pltpu.make_async_copy(u_hbm.at[idxs.at[j1]], ubuf1, gu1).wait()
      pltpu.make_async_copy(z_hbm.at[idxd.at[j1]], zbuf1, gz1).wait()
      pltpu.async_copy(u_hbm.at[idxs.at[j0 + 2]], ubuf0, gu0)
      pltpu.async_copy(z_hbm.at[idxd.at[j0 + 2]], zbuf0, gz0)

      @pl.when(i > 0)
      def _():
        pltpu.make_async_copy(obuf1, out_hbm.at[pl.ds(off1, CH)], wo1).wait()

      dot_chunk(ubuf1, zbuf1, obuf1)
      pltpu.async_copy(obuf1, out_hbm.at[pl.ds(off1, CH)], wo1)
      return carry

    lax.fori_loop(0, NCHUNK // 2, step, 0)

    last = NCHUNK - 1
    offl = pl.multiple_of(base + last * CH, 8)
    pltpu.make_async_copy(u_hbm.at[idxs.at[last]], ubuf0, gu0).wait()
    pltpu.make_async_copy(z_hbm.at[idxd.at[last]], zbuf0, gz0).wait()
    pltpu.make_async_copy(obuf0, out_hbm.at[pl.ds(offl, CH)], wo0).wait()
    dot_chunk(ubuf0, zbuf0, obuf0)
    pltpu.sync_copy(obuf0, out_hbm.at[pl.ds(offl, CH)])
    pltpu.make_async_copy(obuf1, out_hbm.at[pl.ds(offl, CH)], wo1).wait()

  return k(u, z, srcr, dstr, bb16)


_NBLK = 1000
_GRID = N // _NBLK


def _dinv_from(dega_ref, degb_ref):
  deg = 1.0 + dega_ref[:, 0:1] + degb_ref[:, 0:1]
  return lax.rsqrt(deg)


def _tc_a(x, W1, dega, degb):
  def body(x_ref, w_ref, da_ref, db_ref, q_ref):
    dinv = _dinv_from(da_ref, db_ref)
    p = jnp.dot(x_ref[:, :], w_ref[:, :], preferred_element_type=jnp.float32)
    q_ref[:, :] = dinv * p

  return pl.pallas_call(
      body,
      grid=(_GRID,),
      in_specs=[
          pl.BlockSpec((_NBLK, IN_CH), lambda i: (i, 0)),
          pl.BlockSpec((IN_CH, HID), lambda i: (0, 0)),
          pl.BlockSpec((_NBLK, 16), lambda i: (i, 0)),
          pl.BlockSpec((_NBLK, 16), lambda i: (i, 0)),
      ],
      out_specs=pl.BlockSpec((_NBLK, HID), lambda i: (i, 0)),
      out_shape=jax.ShapeDtypeStruct((N, HID), jnp.float32),
  )(x, W1, dega, degb)


def _tc_c(s1a, s1b, q1, dega, degb, b1, W2):
  def body(a_ref, b_ref, q_ref, da_ref, db_ref, bias_ref, w_ref, out_ref):
    dinv = _dinv_from(da_ref, db_ref)
    h = dinv * (a_ref[:, :] + b_ref[:, :] + q_ref[:, :]) + bias_ref[:, :]
    h = jnp.maximum(h, 0.0)
    p2 = jnp.dot(h, w_ref[:, :], preferred_element_type=jnp.float32)
    out_ref[:, :] = dinv * p2

  return pl.pallas_call(
      body,
      grid=(_GRID,),
      in_specs=[
          pl.BlockSpec((_NBLK, HID), lambda i: (i, 0)),
          pl.BlockSpec((_NBLK, HID), lambda i: (i, 0)),
          pl.BlockSpec((_NBLK, HID), lambda i: (i, 0)),
          pl.BlockSpec((_NBLK, 16), lambda i: (i, 0)),
          pl.BlockSpec((_NBLK, 16), lambda i: (i, 0)),
          pl.BlockSpec((1, HID), lambda i: (0, 0)),
          pl.BlockSpec((HID, OUT_CH), lambda i: (0, 0)),
      ],
      out_specs=pl.BlockSpec((_NBLK, OUT_CH), lambda i: (i, 0)),
      out_shape=jax.ShapeDtypeStruct((N, OUT_CH), jnp.float32),
  )(s1a, s1b, q1, dega, degb, b1, W2)


def _tc_e(s2a, s2b, q2, dega, degb, b2, Wb0):
  def body(a_ref, b_ref, q_ref, da_ref, db_ref, bias_ref, w_ref,
           z_ref, u_ref):
    dinv = _dinv_from(da_ref, db_ref)
    z = dinv * (a_ref[:, :] + b_ref[:, :] + q_ref[:, :]) + bias_ref[:, :]
    z_ref[:, :] = z
    u_ref[:, :] = jnp.dot(z, w_ref[:, :], preferred_element_type=jnp.float32)

  return pl.pallas_call(
      body,
      grid=(_GRID,),
      in_specs=[
          pl.BlockSpec((_NBLK, OUT_CH), lambda i: (i, 0)),
          pl.BlockSpec((_NBLK, OUT_CH), lambda i: (i, 0)),
          pl.BlockSpec((_NBLK, OUT_CH), lambda i: (i, 0)),
          pl.BlockSpec((_NBLK, 16), lambda i: (i, 0)),
          pl.BlockSpec((_NBLK, 16), lambda i: (i, 0)),
          pl.BlockSpec((1, OUT_CH), lambda i: (0, 0)),
          pl.BlockSpec((OUT_CH, OUT_CH), lambda i: (0, 0)),
      ],
      out_specs=[
          pl.BlockSpec((_NBLK, OUT_CH), lambda i: (i, 0)),
          pl.BlockSpec((_NBLK, OUT_CH), lambda i: (i, 0)),
      ],
      out_shape=[
          jax.ShapeDtypeStruct((N, OUT_CH), jnp.float32),
          jax.ShapeDtypeStruct((N, OUT_CH), jnp.float32),
      ],
  )(s2a, s2b, q2, dega, degb, b2, Wb0)


def kernel(x, edge_index, W1, b1, W2, b2, Wb, bb):
  srcr = edge_index[0].reshape(NW, NCHUNK, CH)
  dstr = edge_index[1].reshape(NW, NCHUNK, CH)

  deg2 = _sc_deg(dstr)
  dega, degb = deg2[0], deg2[1]

  q1 = _tc_a(x, W1, dega, degb)
  s1 = _sc_spmm(q1, srcr, dstr, HID)
  q2 = _tc_c(s1[0], s1[1], q1, dega, degb, b1.reshape(1, HID), W2)
  s2 = _sc_spmm(q2, srcr, dstr, OUT_CH)
  z, u = _tc_e(s2[0], s2[1], q2, dega, degb, b2.reshape(1, OUT_CH), Wb[0])
  bb16 = jnp.broadcast_to(bb.reshape(1), (16,))
  return _sc_edge_decode(u, z, srcr, dstr, bb16).reshape(E, 1)


# speedup vs baseline: 22.9173x; 1.0675x over previous
import functools

import jax
import jax.numpy as jnp
from jax import lax
from jax.experimental import pallas as pl
from jax.experimental.pallas import tpu as pltpu
from jax.experimental.pallas import tpu_sc as plsc

N = 10000
E = 320000
IN_CH = 128
HID = 128
OUT_CH = 64

NC = 2
NS = 16
NW = NC * NS
EPW = E // NW
CH = 80
NCHUNK = EPW // CH
ROWB = 80
NROWCH = N // ROWB

_MESH = plsc.VectorSubcoreMesh(
    core_axis_name="c", subcore_axis_name="s", num_cores=NC, num_subcores=NS)


def _fill(buf, rows, width, value):
  vec = jnp.full((16,), value, jnp.float32)

  def body(r, carry):
    for j in range(width // 16):
      buf[r, pl.ds(j * 16, 16)] = vec
    return carry

  lax.fori_loop(0, rows, body, 0)


_DEGW = 8


def _sc_deg(eidx):
  @functools.partial(
      pl.kernel,
      out_type=jax.ShapeDtypeStruct((NC, N, 16), jnp.float32),
      mesh=_MESH,
      compiler_params=pltpu.CompilerParams(use_tc_tiling_on_sc=False),
      scratch_types=[
          pltpu.VMEM((NCHUNK, CH), jnp.int32),
          pltpu.VMEM((ROWB, 16), jnp.float32),
          pltpu.VMEM((ROWB, 16), jnp.float32),
          pltpu.VMEM_SHARED((N, 16), jnp.float32),
          pltpu.SemaphoreType.DMA,
          pltpu.SemaphoreType.DMA,
      ],
  )
  def k(e_hbm, out_hbm, idxd, zrows, ones, acc, psem, ssem):
    c = lax.axis_index("c")
    s = lax.axis_index("s")
    gid = c * NS + s

    cp_idx = pltpu.async_copy(e_hbm.at[1, gid], idxd, psem)
    _fill(zrows, ROWB, 16, 0.0)
    _fill(ones, ROWB, 16, 1.0)

    def zacc(j, carry):
      @pl.when(lax.rem(j, NS) == s)
      def _():
        pltpu.sync_copy(zrows, acc.at[pl.ds(j * ROWB, ROWB)])
      return carry

    lax.fori_loop(0, NROWCH, zacc, 0)
    cp_idx.wait()
    plsc.subcore_barrier()

    def step(j, carry):
      pltpu.async_copy(ones, acc.at[idxd.at[j]], ssem, add=True)

      @pl.when(j >= _DEGW)
      def _():
        pltpu.make_async_copy(ones, acc.at[idxd.at[j]], ssem).wait()
      return carry

    lax.fori_loop(0, NCHUNK, step, 0)

    def drain(j, carry):
      pltpu.make_async_copy(ones, acc.at[idxd.at[0]], ssem).wait()
      return carry

    lax.fori_loop(0, _DEGW, drain, 0)
    plsc.subcore_barrier()

    def cpout(j, carry):
      @pl.when(lax.rem(j, NS) == s)
      def _():
        pltpu.sync_copy(acc.at[pl.ds(j * ROWB, ROWB)], zrows)
        pltpu.sync_copy(zrows, out_hbm.at[c, pl.ds(j * ROWB, ROWB)])
      return carry

    lax.fori_loop(0, NROWCH, cpout, 0)

  return k(eidx)


_NB = 2


def _sc_spmm(table, eidx, width):
  @functools.partial(
      pl.kernel,
      out_type=jax.ShapeDtypeStruct((NC, N, width), jnp.float32),
      mesh=_MESH,
      compiler_params=pltpu.CompilerParams(use_tc_tiling_on_sc=False),
      scratch_types=[
          pltpu.VMEM((NCHUNK, CH), jnp.int32),
          pltpu.VMEM((NCHUNK, CH), jnp.int32),
      ] + [pltpu.VMEM((CH, width), jnp.float32)] * _NB + [
          pltpu.VMEM_SHARED((N, width), jnp.float32),
          pltpu.SemaphoreType.DMA,
      ] + [pltpu.SemaphoreType.DMA] * (2 * _NB),
  )
  def k(table_hbm, e_hbm, out_hbm, idxs, idxd, *bufs):
    rows = bufs[:_NB]
    acc = bufs[_NB]
    psem = bufs[_NB + 1]
    g = bufs[_NB + 2:_NB + 2 + _NB]
    st = bufs[_NB + 2 + _NB:]
    c = lax.axis_index("c")
    s = lax.axis_index("s")
    gid = c * NS + s

    cp_si = pltpu.async_copy(e_hbm.at[0, gid], idxs, psem)
    cp_di = pltpu.async_copy(e_hbm.at[1, gid], idxd, psem)

    _fill(rows[0], CH, width, 0.0)

    def zacc(j, carry):
      @pl.when(lax.rem(j, NS) == s)
      def _():
        pltpu.sync_copy(rows[0], acc.at[pl.ds(j * ROWB, ROWB)])
      return carry

    lax.fori_loop(0, NROWCH, zacc, 0)
    cp_si.wait()
    cp_di.wait()
    for b in range(_NB - 1):
      pltpu.async_copy(table_hbm.at[idxs.at[b]], rows[b], g[b])
    plsc.subcore_barrier()

    def step(i, carry):
      for b in range(_NB):
        j = _NB * i + b
        pltpu.make_async_copy(table_hbm.at[idxs.at[j]], rows[b], g[b]).wait()
        bn = (b + _NB - 1) % _NB
        if b == 0:
          @pl.when(i > 0)
          def _():
            pltpu.make_async_copy(rows[bn], acc.at[idxd.at[j]], st[bn]).wait()
          pltpu.async_copy(table_hbm.at[idxs.at[j + _NB - 1]], rows[bn], g[bn])
        else:
          pltpu.make_async_copy(rows[bn], acc.at[idxd.at[j]], st[bn]).wait()

          @pl.when(i < NCHUNK // _NB - 1)
          def _():
            pltpu.async_copy(table_hbm.at[idxs.at[j + _NB - 1]], rows[bn], g[bn])
        pltpu.async_copy(rows[b], acc.at[idxd.at[j]], st[b], add=True)
      return carry

    lax.fori_loop(0, NCHUNK // _NB, step, 0)
    pltpu.make_async_copy(rows[_NB - 1], acc.at[idxd.at[0]],
                          st[_NB - 1]).wait()
    plsc.subcore_barrier()

    def cpout(j, carry):
      @pl.when(lax.rem(j, NS) == s)
      def _():
        pltpu.sync_copy(acc.at[pl.ds(j * ROWB, ROWB)], rows[0])
        pltpu.sync_copy(rows[0], out_hbm.at[c, pl.ds(j * ROWB, ROWB)])
      return carry

    lax.fori_loop(0, NROWCH, cpout, 0)

  return k(table, eidx)


_NQ = OUT_CH // 16


def _sc_edge_decode(u, z, eidx, bb16):
  @functools.partial(
      pl.kernel,
      out_type=jax.ShapeDtypeStruct((E,), jnp.float32),
      mesh=_MESH,
      compiler_params=pltpu.CompilerParams(
          use_tc_tiling_on_sc=False, needs_layout_passes=False),
      scratch_types=[
          pltpu.VMEM((NCHUNK, CH), jnp.int32),
          pltpu.VMEM((NCHUNK, CH), jnp.int32),
          pltpu.VMEM((CH, OUT_CH), jnp.float32),
          pltpu.VMEM((CH, OUT_CH), jnp.float32),
          pltpu.VMEM((CH, OUT_CH), jnp.float32),
          pltpu.VMEM((CH, OUT_CH), jnp.float32),
          pltpu.VMEM((CH,), jnp.float32),
          pltpu.VMEM((CH,), jnp.float32),
          pltpu.VMEM((16,), jnp.float32),
      ] + [pltpu.SemaphoreType.DMA] * 7,
  )
  def k(u_hbm, z_hbm, e_hbm, bb_hbm, out_hbm,
        idxs, idxd, ubuf0, zbuf0, ubuf1, zbuf1, obuf0, obuf1, bbv,
        psem, gu0, gz0, gu1, gz1, wo0, wo1):
    c = lax.axis_index("c")
    s = lax.axis_index("s")
    gid = c * NS + s
    base = gid * EPW

    cp_si = pltpu.async_copy(e_hbm.at[0, gid], idxs, psem)
    cp_di = pltpu.async_copy(e_hbm.at[1, gid], idxd, psem)
    pltpu.sync_copy(bb_hbm, bbv)
    bias = bbv[...]
    lane = lax.iota(jnp.int32, 16)
    cp_si.wait()
    cp_di.wait()
    pltpu.async_copy(u_hbm.at[idxs.at[0]], ubuf0, gu0)
    pltpu.async_copy(z_hbm.at[idxd.at[0]], zbuf0, gz0)

    def dot_chunk(ubuf, zbuf, obuf):
      def grp(g, carry):
        res = jnp.zeros((16,), jnp.float32)
        for e in range(16):
          row = g * 16 + e
          acc = ubuf[row, pl.ds(0, 16)] * zbuf[row, pl.ds(0, 16)]
          for q in range(1, _NQ):
            acc = acc + ubuf[row, pl.ds(q * 16, 16)] * zbuf[row, pl.ds(q * 16, 16)]
          res = jnp.where(lane == e, jnp.full((16,), jnp.sum(acc)), res)
        obuf[pl.ds(g * 16, 16)] = 1.0 / (1.0 + jnp.exp(-(res + bias)))
        return carry

      lax.fori_loop(0, CH // 16, grp, 0)

    def step(i, carry):
      j0 = 2 * i
      j1 = 2 * i + 1
      off0 = pl.multiple_of(base + j0 * CH, 8)
      off1 = pl.multiple_of(base + j1 * CH, 8)
      pltpu.make_async_copy(u_hbm.at[idxs.at[j0]], ubuf0, gu0).wait()
      pltpu.make_async_copy(z_hbm.at[idxd.at[j0]], zbuf0, gz0).wait()
      pltpu.async_copy(u_hbm.at[idxs.at[j1]], ubuf1, gu1)
      pltpu.async_copy(z_hbm.at[idxd.at[j1]], zbuf1, gz1)

      @pl.when(i > 0)
      def _():
        pltpu.make_async_copy(obuf0, out_hbm.at[pl.ds(off0, CH)], wo0).wait()

      dot_chunk(ubuf0, zbuf0, obuf0)
      pltpu.async_copy(obuf0, out_hbm.at[pl.ds(off0, CH)], wo0)
      pltpu.make_async_copy(u_hbm.at[idxs.at[j1]], ubuf1, gu1).wait()
      pltpu.make_async_copy(z_hbm.at[idxd.at[j1]], zbuf1, gz1).wait()
      pltpu.async_copy(u_hbm.at[idxs.at[j0 + 2]], ubuf0, gu0)
      pltpu.async_copy(z_hbm.at[idxd.at[j0 + 2]], zbuf0, gz0)

      @pl.when(i > 0)
      def _():
        pltpu.make_async_copy(obuf1, out_hbm.at[pl.ds(off1, CH)], wo1).wait()

      dot_chunk(ubuf1, zbuf1, obuf1)
      pltpu.async_copy(obuf1, out_hbm.at[pl.ds(off1, CH)], wo1)
      return carry

    lax.fori_loop(0, NCHUNK // 2, step, 0)

    last = NCHUNK - 1
    offl = pl.multiple_of(base + last * CH, 8)
    pltpu.make_async_copy(u_hbm.at[idxs.at[last]], ubuf0, gu0).wait()
    pltpu.make_async_copy(z_hbm.at[idxd.at[last]], zbuf0, gz0).wait()
    pltpu.make_async_copy(obuf0, out_hbm.at[pl.ds(offl, CH)], wo0).wait()
    dot_chunk(ubuf0, zbuf0, obuf0)
    pltpu.sync_copy(obuf0, out_hbm.at[pl.ds(offl, CH)])
    pltpu.make_async_copy(obuf1, out_hbm.at[pl.ds(offl, CH)], wo1).wait()

  return k(u, z, eidx, bb16)


_NBLK = 1000
_GRID = N // _NBLK


def _dinv_from(deg_ref):
  deg = 1.0 + deg_ref[0, :, 0:1] + deg_ref[1, :, 0:1]
  return lax.rsqrt(deg)


_DEGSPEC = pl.BlockSpec((NC, _NBLK, 16), lambda i: (0, i, 0))


def _tc_a(x, W1, deg2):
  def body(x_ref, w_ref, d_ref, q_ref):
    dinv = _dinv_from(d_ref)
    p = jnp.dot(x_ref[:, :], w_ref[:, :], preferred_element_type=jnp.float32)
    q_ref[:, :] = dinv * p

  return pl.pallas_call(
      body,
      grid=(_GRID,),
      in_specs=[
          pl.BlockSpec((_NBLK, IN_CH), lambda i: (i, 0)),
          pl.BlockSpec((IN_CH, HID), lambda i: (0, 0)),
          _DEGSPEC,
      ],
      out_specs=pl.BlockSpec((_NBLK, HID), lambda i: (i, 0)),
      out_shape=jax.ShapeDtypeStruct((N, HID), jnp.float32),
  )(x, W1, deg2)


def _tc_c(s1, q1, deg2, b1, W2):
  def body(s_ref, q_ref, d_ref, bias_ref, w_ref, out_ref):
    dinv = _dinv_from(d_ref)
    h = dinv * (s_ref[0] + s_ref[1] + q_ref[:, :]) + bias_ref[:, :]
    h = jnp.maximum(h, 0.0)
    p2 = jnp.dot(h, w_ref[:, :], preferred_element_type=jnp.float32)
    out_ref[:, :] = dinv * p2

  return pl.pallas_call(
      body,
      grid=(_GRID,),
      in_specs=[
          pl.BlockSpec((NC, _NBLK, HID), lambda i: (0, i, 0)),
          pl.BlockSpec((_NBLK, HID), lambda i: (i, 0)),
          _DEGSPEC,
          pl.BlockSpec((1, HID), lambda i: (0, 0)),
          pl.BlockSpec((HID, OUT_CH), lambda i: (0, 0)),
      ],
      out_specs=pl.BlockSpec((_NBLK, OUT_CH), lambda i: (i, 0)),
      out_shape=jax.ShapeDtypeStruct((N, OUT_CH), jnp.float32),
  )(s1, q1, deg2, b1, W2)


def _tc_e(s2, q2, deg2, b2, Wb0):
  def body(s_ref, q_ref, d_ref, bias_ref, w_ref, z_ref, u_ref):
    dinv = _dinv_from(d_ref)
    z = dinv * (s_ref[0] + s_ref[1] + q_ref[:, :]) + bias_ref[:, :]
    z_ref[:, :] = z
    u_ref[:, :] = jnp.dot(z, w_ref[:, :], preferred_element_type=jnp.float32)

  return pl.pallas_call(
      body,
      grid=(_GRID,),
      in_specs=[
          pl.BlockSpec((NC, _NBLK, OUT_CH), lambda i: (0, i, 0)),
          pl.BlockSpec((_NBLK, OUT_CH), lambda i: (i, 0)),
          _DEGSPEC,
          pl.BlockSpec((1, OUT_CH), lambda i: (0, 0)),
          pl.BlockSpec((OUT_CH, OUT_CH), lambda i: (0, 0)),
      ],
      out_specs=[
          pl.BlockSpec((_NBLK, OUT_CH), lambda i: (i, 0)),
          pl.BlockSpec((_NBLK, OUT_CH), lambda i: (i, 0)),
      ],
      out_shape=[
          jax.ShapeDtypeStruct((N, OUT_CH), jnp.float32),
          jax.ShapeDtypeStruct((N, OUT_CH), jnp.float32),
      ],
  )(s2, q2, deg2, b2, Wb0)


def kernel(x, edge_index, W1, b1, W2, b2, Wb, bb):
  eidx = edge_index.reshape(2, NW, NCHUNK, CH)

  deg2 = _sc_deg(eidx)
  q1 = _tc_a(x, W1, deg2)
  s1 = _sc_spmm(q1, eidx, HID)
  q2 = _tc_c(s1, q1, deg2, b1.reshape(1, HID), W2)
  s2 = _sc_spmm(q2, eidx, OUT_CH)
  z, u = _tc_e(s2, q2, deg2, b2.reshape(1, OUT_CH), Wb[0])
  bb16 = jnp.broadcast_to(bb.reshape(1), (16,))
  return _sc_edge_decode(u, z, eidx, bb16).reshape(E, 1)


# speedup vs baseline: 23.2482x; 1.0144x over previous
import functools

import jax
import jax.numpy as jnp
from jax import lax
from jax.experimental import pallas as pl
from jax.experimental.pallas import tpu as pltpu
from jax.experimental.pallas import tpu_sc as plsc

N = 10000
E = 320000
IN_CH = 128
HID = 128
OUT_CH = 64

NC = 2
NS = 16
NW = NC * NS
EPW = E // NW
CH = 80
NCHUNK = EPW // CH
ROWB = 80
NROWCH = N // ROWB

_MESH = plsc.VectorSubcoreMesh(
    core_axis_name="c", subcore_axis_name="s", num_cores=NC, num_subcores=NS)


def _fill(buf, rows, width, value):
  vec = jnp.full((16,), value, jnp.float32)

  def body(r, carry):
    for j in range(width // 16):
      buf[r, pl.ds(j * 16, 16)] = vec
    return carry

  lax.fori_loop(0, rows, body, 0)


_DEGW = 8


def _sc_deg(eidx):
  @functools.partial(
      pl.kernel,
      out_type=jax.ShapeDtypeStruct((NC, N, 16), jnp.float32),
      mesh=_MESH,
      compiler_params=pltpu.CompilerParams(use_tc_tiling_on_sc=False),
      scratch_types=[
          pltpu.VMEM((NCHUNK, CH), jnp.int32),
          pltpu.VMEM((ROWB, 16), jnp.float32),
          pltpu.VMEM((ROWB, 16), jnp.float32),
          pltpu.VMEM_SHARED((N, 16), jnp.float32),
          pltpu.SemaphoreType.DMA,
          pltpu.SemaphoreType.DMA,
      ],
  )
  def k(e_hbm, out_hbm, idxd, zrows, ones, acc, psem, ssem):
    c = lax.axis_index("c")
    s = lax.axis_index("s")
    gid = c * NS + s

    cp_idx = pltpu.async_copy(e_hbm.at[1, gid], idxd, psem)
    _fill(zrows, ROWB, 16, 0.0)
    _fill(ones, ROWB, 16, 1.0)

    def zacc(j, carry):
      @pl.when(lax.rem(j, NS) == s)
      def _():
        pltpu.sync_copy(zrows, acc.at[pl.ds(j * ROWB, ROWB)])
      return carry

    lax.fori_loop(0, NROWCH, zacc, 0)
    cp_idx.wait()
    plsc.subcore_barrier()

    def step(j, carry):
      pltpu.async_copy(ones, acc.at[idxd.at[j]], ssem, add=True)

      @pl.when(j >= _DEGW)
      def _():
        pltpu.make_async_copy(ones, acc.at[idxd.at[j]], ssem).wait()
      return carry

    lax.fori_loop(0, NCHUNK, step, 0)

    def drain(j, carry):
      pltpu.make_async_copy(ones, acc.at[idxd.at[0]], ssem).wait()
      return carry

    lax.fori_loop(0, _DEGW, drain, 0)
    plsc.subcore_barrier()

    def cpout(j, carry):
      @pl.when(lax.rem(j, NS) == s)
      def _():
        pltpu.sync_copy(acc.at[pl.ds(j * ROWB, ROWB)], zrows)
        pltpu.sync_copy(zrows, out_hbm.at[c, pl.ds(j * ROWB, ROWB)])
      return carry

    lax.fori_loop(0, NROWCH, cpout, 0)

  return k(eidx)


_NB = 2


def _sc_spmm(table, eidx, width):
  @functools.partial(
      pl.kernel,
      out_type=jax.ShapeDtypeStruct((NC, N, width), jnp.float32),
      mesh=_MESH,
      compiler_params=pltpu.CompilerParams(use_tc_tiling_on_sc=False),
      scratch_types=[
          pltpu.VMEM((NCHUNK, CH), jnp.int32),
          pltpu.VMEM((NCHUNK, CH), jnp.int32),
      ] + [pltpu.VMEM((CH, width), jnp.float32)] * _NB + [
          pltpu.VMEM_SHARED((N, width), jnp.float32),
          pltpu.SemaphoreType.DMA,
      ] + [pltpu.SemaphoreType.DMA] * (2 * _NB),
  )
  def k(table_hbm, e_hbm, out_hbm, idxs, idxd, *bufs):
    rows = bufs[:_NB]
    acc = bufs[_NB]
    psem = bufs[_NB + 1]
    g = bufs[_NB + 2:_NB + 2 + _NB]
    st = bufs[_NB + 2 + _NB:]
    c = lax.axis_index("c")
    s = lax.axis_index("s")
    gid = c * NS + s

    cp_si = pltpu.async_copy(e_hbm.at[0, gid], idxs, psem)
    cp_di = pltpu.async_copy(e_hbm.at[1, gid], idxd, psem)

    _fill(rows[0], CH, width, 0.0)

    def zacc(j, carry):
      @pl.when(lax.rem(j, NS) == s)
      def _():
        pltpu.sync_copy(rows[0], acc.at[pl.ds(j * ROWB, ROWB)])
      return carry

    lax.fori_loop(0, NROWCH, zacc, 0)
    cp_si.wait()
    cp_di.wait()
    for b in range(_NB - 1):
      pltpu.async_copy(table_hbm.at[idxs.at[b]], rows[b], g[b])
    plsc.subcore_barrier()

    def step(i, carry):
      for b in range(_NB):
        j = _NB * i + b
        pltpu.make_async_copy(table_hbm.at[idxs.at[j]], rows[b], g[b]).wait()
        bn = (b + _NB - 1) % _NB
        if b == 0:
          @pl.when(i > 0)
          def _():
            pltpu.make_async_copy(rows[bn], acc.at[idxd.at[j]], st[bn]).wait()
          pltpu.async_copy(table_hbm.at[idxs.at[j + _NB - 1]], rows[bn], g[bn])
        else:
          pltpu.make_async_copy(rows[bn], acc.at[idxd.at[j]], st[bn]).wait()

          @pl.when(i < NCHUNK // _NB - 1)
          def _():
            pltpu.async_copy(table_hbm.at[idxs.at[j + _NB - 1]], rows[bn], g[bn])
        pltpu.async_copy(rows[b], acc.at[idxd.at[j]], st[b], add=True)
      return carry

    lax.fori_loop(0, NCHUNK // _NB, step, 0)
    pltpu.make_async_copy(rows[_NB - 1], acc.at[idxd.at[0]],
                          st[_NB - 1]).wait()
    plsc.subcore_barrier()

    def cpout(j, carry):
      @pl.when(lax.rem(j, NS) == s)
      def _():
        pltpu.sync_copy(acc.at[pl.ds(j * ROWB, ROWB)], rows[0])
        pltpu.sync_copy(rows[0], out_hbm.at[c, pl.ds(j * ROWB, ROWB)])
      return carry

    lax.fori_loop(0, NROWCH, cpout, 0)

  return k(table, eidx)


_NQ = OUT_CH // 16


def _sc_edge_decode(u, z, eidx, bb16):
  @functools.partial(
      pl.kernel,
      out_type=jax.ShapeDtypeStruct((E,), jnp.float32),
      mesh=_MESH,
      compiler_params=pltpu.CompilerParams(
          use_tc_tiling_on_sc=False, needs_layout_passes=False),
      scratch_types=[
          pltpu.VMEM((NCHUNK, CH), jnp.int32),
          pltpu.VMEM((NCHUNK, CH), jnp.int32),
          pltpu.VMEM((CH, OUT_CH), jnp.float32),
          pltpu.VMEM((CH, OUT_CH), jnp.float32),
          pltpu.VMEM((CH, OUT_CH), jnp.float32),
          pltpu.VMEM((CH, OUT_CH), jnp.float32),
          pltpu.VMEM((CH,), jnp.float32),
          pltpu.VMEM((CH,), jnp.float32),
          pltpu.VMEM((16,), jnp.float32),
      ] + [pltpu.SemaphoreType.DMA] * 7,
  )
  def k(u_hbm, z_hbm, e_hbm, bb_hbm, out_hbm,
        idxs, idxd, ubuf0, zbuf0, ubuf1, zbuf1, obuf0, obuf1, bbv,
        psem, gu0, gz0, gu1, gz1, wo0, wo1):
    c = lax.axis_index("c")
    s = lax.axis_index("s")
    gid = c * NS + s
    base = gid * EPW

    cp_si = pltpu.async_copy(e_hbm.at[0, gid], idxs, psem)
    cp_di = pltpu.async_copy(e_hbm.at[1, gid], idxd, psem)
    pltpu.sync_copy(bb_hbm, bbv)
    bias = bbv[...]
    lane = lax.iota(jnp.int32, 16)
    cp_si.wait()
    cp_di.wait()
    pltpu.async_copy(u_hbm.at[idxs.at[0]], ubuf0, gu0)
    pltpu.async_copy(z_hbm.at[idxd.at[0]], zbuf0, gz0)

    def dot_chunk(ubuf, zbuf, obuf):
      def grp(g, carry):
        res = jnp.zeros((16,), jnp.float32)
        for e in range(16):
          row = g * 16 + e
          acc = ubuf[row, pl.ds(0, 16)] * zbuf[row, pl.ds(0, 16)]
          for q in range(1, _NQ):
            acc = acc + ubuf[row, pl.ds(q * 16, 16)] * zbuf[row, pl.ds(q * 16, 16)]
          res = jnp.where(lane == e, jnp.full((16,), jnp.sum(acc)), res)
        obuf[pl.ds(g * 16, 16)] = 1.0 / (1.0 + jnp.exp(-(res + bias)))
        return carry

      lax.fori_loop(0, CH // 16, grp, 0)

    def step(i, carry):
      j0 = 2 * i
      j1 = 2 * i + 1
      off0 = pl.multiple_of(base + j0 * CH, 8)
      off1 = pl.multiple_of(base + j1 * CH, 8)
      pltpu.make_async_copy(u_hbm.at[idxs.at[j0]], ubuf0, gu0).wait()
      pltpu.make_async_copy(z_hbm.at[idxd.at[j0]], zbuf0, gz0).wait()
      pltpu.async_copy(u_hbm.at[idxs.at[j1]], ubuf1, gu1)
      pltpu.async_copy(z_hbm.at[idxd.at[j1]], zbuf1, gz1)

      @pl.when(i > 0)
      def _():
        pltpu.make_async_copy(obuf0, out_hbm.at[pl.ds(off0, CH)], wo0).wait()

      dot_chunk(ubuf0, zbuf0, obuf0)
      pltpu.async_copy(obuf0, out_hbm.at[pl.ds(off0, CH)], wo0)
      pltpu.make_async_copy(u_hbm.at[idxs.at[j1]], ubuf1, gu1).wait()
      pltpu.make_async_copy(z_hbm.at[idxd.at[j1]], zbuf1, gz1).wait()
      pltpu.async_copy(u_hbm.at[idxs.at[j0 + 2]], ubuf0, gu0)
      pltpu.async_copy(z_hbm.at[idxd.at[j0 + 2]], zbuf0, gz0)

      @pl.when(i > 0)
      def _():
        pltpu.make_async_copy(obuf1, out_hbm.at[pl.ds(off1, CH)], wo1).wait()

      dot_chunk(ubuf1, zbuf1, obuf1)
      pltpu.async_copy(obuf1, out_hbm.at[pl.ds(off1, CH)], wo1)
      return carry

    lax.fori_loop(0, NCHUNK // 2, step, 0)

    last = NCHUNK - 1
    offl = pl.multiple_of(base + last * CH, 8)
    pltpu.make_async_copy(u_hbm.at[idxs.at[last]], ubuf0, gu0).wait()
    pltpu.make_async_copy(z_hbm.at[idxd.at[last]], zbuf0, gz0).wait()
    pltpu.make_async_copy(obuf0, out_hbm.at[pl.ds(offl, CH)], wo0).wait()
    dot_chunk(ubuf0, zbuf0, obuf0)
    pltpu.sync_copy(obuf0, out_hbm.at[pl.ds(offl, CH)])
    pltpu.make_async_copy(obuf1, out_hbm.at[pl.ds(offl, CH)], wo1).wait()

  return k(u, z, eidx, bb16)


_NBLK = 2000
_GRID = N // _NBLK


def _dinv_from(deg_ref):
  deg = 1.0 + deg_ref[0, :, 0:1] + deg_ref[1, :, 0:1]
  return lax.rsqrt(deg)


_DEGSPEC = pl.BlockSpec((NC, _NBLK, 16), lambda i: (0, i, 0))


def _tc_a(x, W1, deg2):
  def body(x_ref, w_ref, d_ref, q_ref):
    dinv = _dinv_from(d_ref)
    p = jnp.dot(x_ref[:, :], w_ref[:, :], preferred_element_type=jnp.float32)
    q_ref[:, :] = dinv * p

  return pl.pallas_call(
      body,
      grid=(_GRID,),
      in_specs=[
          pl.BlockSpec((_NBLK, IN_CH), lambda i: (i, 0)),
          pl.BlockSpec((IN_CH, HID), lambda i: (0, 0)),
          _DEGSPEC,
      ],
      out_specs=pl.BlockSpec((_NBLK, HID), lambda i: (i, 0)),
      out_shape=jax.ShapeDtypeStruct((N, HID), jnp.float32),
  )(x, W1, deg2)


def _tc_c(s1, q1, deg2, b1, W2):
  def body(s_ref, q_ref, d_ref, bias_ref, w_ref, out_ref):
    dinv = _dinv_from(d_ref)
    h = dinv * (s_ref[0] + s_ref[1] + q_ref[:, :]) + bias_ref[:, :]
    h = jnp.maximum(h, 0.0)
    p2 = jnp.dot(h, w_ref[:, :], preferred_element_type=jnp.float32)
    out_ref[:, :] = dinv * p2

  return pl.pallas_call(
      body,
      grid=(_GRID,),
      in_specs=[
          pl.BlockSpec((NC, _NBLK, HID), lambda i: (0, i, 0)),
          pl.BlockSpec((_NBLK, HID), lambda i: (i, 0)),
          _DEGSPEC,
          pl.BlockSpec((1, HID), lambda i: (0, 0)),
          pl.BlockSpec((HID, OUT_CH), lambda i: (0, 0)),
      ],
      out_specs=pl.BlockSpec((_NBLK, OUT_CH), lambda i: (i, 0)),
      out_shape=jax.ShapeDtypeStruct((N, OUT_CH), jnp.float32),
  )(s1, q1, deg2, b1, W2)


def _tc_e(s2, q2, deg2, b2, Wb0):
  def body(s_ref, q_ref, d_ref, bias_ref, w_ref, z_ref, u_ref):
    dinv = _dinv_from(d_ref)
    z = dinv * (s_ref[0] + s_ref[1] + q_ref[:, :]) + bias_ref[:, :]
    z_ref[:, :] = z
    u_ref[:, :] = jnp.dot(z, w_ref[:, :], preferred_element_type=jnp.float32)

  return pl.pallas_call(
      body,
      grid=(_GRID,),
      in_specs=[
          pl.BlockSpec((NC, _NBLK, OUT_CH), lambda i: (0, i, 0)),
          pl.BlockSpec((_NBLK, OUT_CH), lambda i: (i, 0)),
          _DEGSPEC,
          pl.BlockSpec((1, OUT_CH), lambda i: (0, 0)),
          pl.BlockSpec((OUT_CH, OUT_CH), lambda i: (0, 0)),
      ],
      out_specs=[
          pl.BlockSpec((_NBLK, OUT_CH), lambda i: (i, 0)),
          pl.BlockSpec((_NBLK, OUT_CH), lambda i: (i, 0)),
      ],
      out_shape=[
          jax.ShapeDtypeStruct((N, OUT_CH), jnp.float32),
          jax.ShapeDtypeStruct((N, OUT_CH), jnp.float32),
      ],
  )(s2, q2, deg2, b2, Wb0)


def kernel(x, edge_index, W1, b1, W2, b2, Wb, bb):
  eidx = edge_index.reshape(2, NW, NCHUNK, CH)

  deg2 = _sc_deg(eidx)
  q1 = _tc_a(x, W1, deg2)
  s1 = _sc_spmm(q1, eidx, HID)
  q2 = _tc_c(s1, q1, deg2, b1.reshape(1, HID), W2)
  s2 = _sc_spmm(q2, eidx, OUT_CH)
  z, u = _tc_e(s2, q2, deg2, b2.reshape(1, OUT_CH), Wb[0])
  bb16 = jnp.broadcast_to(bb.reshape(1), (16,))
  return _sc_edge_decode(u, z, eidx, bb16).reshape(E, 1)


# speedup vs baseline: 26.6422x; 1.1460x over previous
import functools

import jax
import jax.numpy as jnp
from jax import lax
from jax.experimental import pallas as pl
from jax.experimental.pallas import tpu as pltpu
from jax.experimental.pallas import tpu_sc as plsc

N = 10000
E = 320000
IN_CH = 128
HID = 128
OUT_CH = 64

NC = 2
NS = 16
NW = NC * NS
EPW = E // NW
CH = 80
NCHUNK = EPW // CH
ROWB = 80
NROWCH = N // ROWB

_MESH = plsc.VectorSubcoreMesh(
    core_axis_name="c", subcore_axis_name="s", num_cores=NC, num_subcores=NS)


def _fill(buf, rows, width, value):
  vec = jnp.full((16,), value, jnp.float32)

  def body(r, carry):
    for j in range(width // 16):
      buf[r, pl.ds(j * 16, 16)] = vec
    return carry

  lax.fori_loop(0, rows, body, 0)


_DEGW = 8


def _sc_deg(eidx):
  @functools.partial(
      pl.kernel,
      out_type=jax.ShapeDtypeStruct((NC, N, 16), jnp.float32),
      mesh=_MESH,
      compiler_params=pltpu.CompilerParams(use_tc_tiling_on_sc=False),
      scratch_types=[
          pltpu.VMEM((NCHUNK, CH), jnp.int32),
          pltpu.VMEM((ROWB, 16), jnp.float32),
          pltpu.VMEM((ROWB, 16), jnp.float32),
          pltpu.VMEM_SHARED((N, 16), jnp.float32),
          pltpu.SemaphoreType.DMA,
          pltpu.SemaphoreType.DMA,
      ],
  )
  def k(e_hbm, out_hbm, idxd, zrows, ones, acc, psem, ssem):
    c = lax.axis_index("c")
    s = lax.axis_index("s")
    gid = c * NS + s

    cp_idx = pltpu.async_copy(e_hbm.at[1, gid], idxd, psem)
    _fill(zrows, ROWB, 16, 0.0)
    _fill(ones, ROWB, 16, 1.0)

    def zacc(j, carry):
      @pl.when(lax.rem(j, NS) == s)
      def _():
        pltpu.sync_copy(zrows, acc.at[pl.ds(j * ROWB, ROWB)])
      return carry

    lax.fori_loop(0, NROWCH, zacc, 0)
    cp_idx.wait()
    plsc.subcore_barrier()

    def step(j, carry):
      pltpu.async_copy(ones, acc.at[idxd.at[j]], ssem, add=True)

      @pl.when(j >= _DEGW)
      def _():
        pltpu.make_async_copy(ones, acc.at[idxd.at[j]], ssem).wait()
      return carry

    lax.fori_loop(0, NCHUNK, step, 0)

    def drain(j, carry):
      pltpu.make_async_copy(ones, acc.at[idxd.at[0]], ssem).wait()
      return carry

    lax.fori_loop(0, _DEGW, drain, 0)
    plsc.subcore_barrier()

    def cpout(j, carry):
      @pl.when(lax.rem(j, NS) == s)
      def _():
        pltpu.sync_copy(acc.at[pl.ds(j * ROWB, ROWB)], zrows)
        pltpu.sync_copy(zrows, out_hbm.at[c, pl.ds(j * ROWB, ROWB)])
      return carry

    lax.fori_loop(0, NROWCH, cpout, 0)

  return k(eidx)


def _sc_spmm(table, eidx, width, _NB):
  @functools.partial(
      pl.kernel,
      out_type=jax.ShapeDtypeStruct((NC, N, width), jnp.float32),
      mesh=_MESH,
      compiler_params=pltpu.CompilerParams(use_tc_tiling_on_sc=False),
      scratch_types=[
          pltpu.VMEM((NCHUNK, CH), jnp.int32),
          pltpu.VMEM((NCHUNK, CH), jnp.int32),
      ] + [pltpu.VMEM((CH, width), jnp.float32)] * _NB + [
          pltpu.VMEM_SHARED((N, width), jnp.float32),
          pltpu.SemaphoreType.DMA,
      ] + [pltpu.SemaphoreType.DMA] * (2 * _NB),
  )
  def k(table_hbm, e_hbm, out_hbm, idxs, idxd, *bufs):
    rows = bufs[:_NB]
    acc = bufs[_NB]
    psem = bufs[_NB + 1]
    g = bufs[_NB + 2:_NB + 2 + _NB]
    st = bufs[_NB + 2 + _NB:]
    c = lax.axis_index("c")
    s = lax.axis_index("s")
    gid = c * NS + s

    cp_si = pltpu.async_copy(e_hbm.at[0, gid], idxs, psem)
    cp_di = pltpu.async_copy(e_hbm.at[1, gid], idxd, psem)

    _fill(rows[0], CH, width, 0.0)

    def zacc(j, carry):
      @pl.when(lax.rem(j, NS) == s)
      def _():
        pltpu.sync_copy(rows[0], acc.at[pl.ds(j * ROWB, ROWB)])
      return carry

    lax.fori_loop(0, NROWCH, zacc, 0)
    cp_si.wait()
    cp_di.wait()
    for b in range(_NB - 1):
      pltpu.async_copy(table_hbm.at[idxs.at[b]], rows[b], g[b])
    plsc.subcore_barrier()

    def step(i, carry):
      for b in range(_NB):
        j = _NB * i + b
        pltpu.make_async_copy(table_hbm.at[idxs.at[j]], rows[b], g[b]).wait()
        bn = (b + _NB - 1) % _NB
        if b == 0:
          @pl.when(i > 0)
          def _():
            pltpu.make_async_copy(rows[bn], acc.at[idxd.at[j]], st[bn]).wait()
          pltpu.async_copy(table_hbm.at[idxs.at[j + _NB - 1]], rows[bn], g[bn])
        else:
          pltpu.make_async_copy(rows[bn], acc.at[idxd.at[j]], st[bn]).wait()

          @pl.when(i < NCHUNK // _NB - 1)
          def _():
            pltpu.async_copy(table_hbm.at[idxs.at[j + _NB - 1]], rows[bn], g[bn])
        pltpu.async_copy(rows[b], acc.at[idxd.at[j]], st[b], add=True)
      return carry

    lax.fori_loop(0, NCHUNK // _NB, step, 0)
    pltpu.make_async_copy(rows[_NB - 1], acc.at[idxd.at[0]],
                          st[_NB - 1]).wait()
    plsc.subcore_barrier()

    def cpout(j, carry):
      @pl.when(lax.rem(j, NS) == s)
      def _():
        pltpu.sync_copy(acc.at[pl.ds(j * ROWB, ROWB)], rows[0])
        pltpu.sync_copy(rows[0], out_hbm.at[c, pl.ds(j * ROWB, ROWB)])
      return carry

    lax.fori_loop(0, NROWCH, cpout, 0)

  return k(table, eidx)


_NQ = OUT_CH // 16


def _sc_edge_decode(u, z, eidx, bb16):
  @functools.partial(
      pl.kernel,
      out_type=jax.ShapeDtypeStruct((E,), jnp.float32),
      mesh=_MESH,
      compiler_params=pltpu.CompilerParams(
          use_tc_tiling_on_sc=False, needs_layout_passes=False),
      scratch_types=[
          pltpu.VMEM((NCHUNK, CH), jnp.int32),
          pltpu.VMEM((NCHUNK, CH), jnp.int32),
          pltpu.VMEM((CH, OUT_CH), jnp.float32),
          pltpu.VMEM((CH, OUT_CH), jnp.float32),
          pltpu.VMEM((CH, OUT_CH), jnp.float32),
          pltpu.VMEM((CH, OUT_CH), jnp.float32),
          pltpu.VMEM((CH,), jnp.float32),
          pltpu.VMEM((CH,), jnp.float32),
          pltpu.VMEM((16,), jnp.float32),
      ] + [pltpu.SemaphoreType.DMA] * 7,
  )
  def k(u_hbm, z_hbm, e_hbm, bb_hbm, out_hbm,
        idxs, idxd, ubuf0, zbuf0, ubuf1, zbuf1, obuf0, obuf1, bbv,
        psem, gu0, gz0, gu1, gz1, wo0, wo1):
    c = lax.axis_index("c")
    s = lax.axis_index("s")
    gid = c * NS + s
    base = gid * EPW

    cp_si = pltpu.async_copy(e_hbm.at[0, gid], idxs, psem)
    cp_di = pltpu.async_copy(e_hbm.at[1, gid], idxd, psem)
    pltpu.sync_copy(bb_hbm, bbv)
    bias = bbv[...]
    lane = lax.iota(jnp.int32, 16)
    cp_si.wait()
    cp_di.wait()
    pltpu.async_copy(u_hbm.at[idxs.at[0]], ubuf0, gu0)
    pltpu.async_copy(z_hbm.at[idxd.at[0]], zbuf0, gz0)

    def dot_chunk(ubuf, zbuf, obuf):
      def grp(g, carry):
        res = jnp.zeros((16,), jnp.float32)
        for e in range(16):
          row = g * 16 + e
          acc = ubuf[row, pl.ds(0, 16)] * zbuf[row, pl.ds(0, 16)]
          for q in range(1, _NQ):
            acc = acc + ubuf[row, pl.ds(q * 16, 16)] * zbuf[row, pl.ds(q * 16, 16)]
          res = jnp.where(lane == e, jnp.full((16,), jnp.sum(acc)), res)
        obuf[pl.ds(g * 16, 16)] = 1.0 / (1.0 + jnp.exp(-(res + bias)))
        return carry

      lax.fori_loop(0, CH // 16, grp, 0)

    def step(i, carry):
      j0 = 2 * i
      j1 = 2 * i + 1
      off0 = pl.multiple_of(base + j0 * CH, 8)
      off1 = pl.multiple_of(base + j1 * CH, 8)
      pltpu.make_async_copy(u_hbm.at[idxs.at[j0]], ubuf0, gu0).wait()
      pltpu.make_async_copy(z_hbm.at[idxd.at[j0]], zbuf0, gz0).wait()
      pltpu.async_copy(u_hbm.at[idxs.at[j1]], ubuf1, gu1)
      pltpu.async_copy(z_hbm.at[idxd.at[j1]], zbuf1, gz1)

      @pl.when(i > 0)
      def _():
        pltpu.make_async_copy(obuf0, out_hbm.at[pl.ds(off0, CH)], wo0).wait()

      dot_chunk(ubuf0, zbuf0, obuf0)
      pltpu.async_copy(obuf0, out_hbm.at[pl.ds(off0, CH)], wo0)
      pltpu.make_async_copy(u_hbm.at[idxs.at[j1]], ubuf1, gu1).wait()
      pltpu.make_async_copy(z_hbm.at[idxd.at[j1]], zbuf1, gz1).wait()
      pltpu.async_copy(u_hbm.at[idxs.at[j0 + 2]], ubuf0, gu0)
      pltpu.async_copy(z_hbm.at[idxd.at[j0 + 2]], zbuf0, gz0)

      @pl.when(i > 0)
      def _():
        pltpu.make_async_copy(obuf1, out_hbm.at[pl.ds(off1, CH)], wo1).wait()

      dot_chunk(ubuf1, zbuf1, obuf1)
      pltpu.async_copy(obuf1, out_hbm.at[pl.ds(off1, CH)], wo1)
      return carry

    lax.fori_loop(0, NCHUNK // 2, step, 0)

    last = NCHUNK - 1
    offl = pl.multiple_of(base + last * CH, 8)
    pltpu.make_async_copy(u_hbm.at[idxs.at[last]], ubuf0, gu0).wait()
    pltpu.make_async_copy(z_hbm.at[idxd.at[last]], zbuf0, gz0).wait()
    pltpu.make_async_copy(obuf0, out_hbm.at[pl.ds(offl, CH)], wo0).wait()
    dot_chunk(ubuf0, zbuf0, obuf0)
    pltpu.sync_copy(obuf0, out_hbm.at[pl.ds(offl, CH)])
    pltpu.make_async_copy(obuf1, out_hbm.at[pl.ds(offl, CH)], wo1).wait()

  return k(u, z, eidx, bb16)


_NBLK = 2000
_GRID = N // _NBLK


def _dinv_from(deg_ref):
  deg = 1.0 + deg_ref[0, :, 0:1] + deg_ref[1, :, 0:1]
  return lax.rsqrt(deg)


_DEGSPEC = pl.BlockSpec((NC, _NBLK, 16), lambda i: (0, i, 0))


def _tc_a(x, W1, deg2):
  def body(x_ref, w_ref, d_ref, q_ref):
    dinv = _dinv_from(d_ref)
    p = jnp.dot(x_ref[:, :], w_ref[:, :], preferred_element_type=jnp.float32)
    q_ref[:, :] = dinv * p

  return pl.pallas_call(
      body,
      grid=(_GRID,),
      in_specs=[
          pl.BlockSpec((_NBLK, IN_CH), lambda i: (i, 0)),
          pl.BlockSpec((IN_CH, HID), lambda i: (0, 0)),
          _DEGSPEC,
      ],
      out_specs=pl.BlockSpec((_NBLK, HID), lambda i: (i, 0)),
      out_shape=jax.ShapeDtypeStruct((N, HID), jnp.float32),
  )(x, W1, deg2)


def _tc_c(s1, q1, deg2, b1, W2):
  def body(s_ref, q_ref, d_ref, bias_ref, w_ref, out_ref):
    dinv = _dinv_from(d_ref)
    h = dinv * (s_ref[0] + s_ref[1] + q_ref[:, :]) + bias_ref[:, :]
    h = jnp.maximum(h, 0.0)
    p2 = jnp.dot(h, w_ref[:, :], preferred_element_type=jnp.float32)
    out_ref[:, :] = dinv * p2

  return pl.pallas_call(
      body,
      grid=(_GRID,),
      in_specs=[
          pl.BlockSpec((NC, _NBLK, HID), lambda i: (0, i, 0)),
          pl.BlockSpec((_NBLK, HID), lambda i: (i, 0)),
          _DEGSPEC,
          pl.BlockSpec((1, HID), lambda i: (0, 0)),
          pl.BlockSpec((HID, OUT_CH), lambda i: (0, 0)),
      ],
      out_specs=pl.BlockSpec((_NBLK, OUT_CH), lambda i: (i, 0)),
      out_shape=jax.ShapeDtypeStruct((N, OUT_CH), jnp.float32),
  )(s1, q1, deg2, b1, W2)


def _tc_e(s2, q2, deg2, b2, Wb0):
  def body(s_ref, q_ref, d_ref, bias_ref, w_ref, z_ref, u_ref):
    dinv = _dinv_from(d_ref)
    z = dinv * (s_ref[0] + s_ref[1] + q_ref[:, :]) + bias_ref[:, :]
    z_ref[:, :] = z
    u_ref[:, :] = jnp.dot(z, w_ref[:, :], preferred_element_type=jnp.float32)

  return pl.pallas_call(
      body,
      grid=(_GRID,),
      in_specs=[
          pl.BlockSpec((NC, _NBLK, OUT_CH), lambda i: (0, i, 0)),
          pl.BlockSpec((_NBLK, OUT_CH), lambda i: (i, 0)),
          _DEGSPEC,
          pl.BlockSpec((1, OUT_CH), lambda i: (0, 0)),
          pl.BlockSpec((OUT_CH, OUT_CH), lambda i: (0, 0)),
      ],
      out_specs=[
          pl.BlockSpec((_NBLK, OUT_CH), lambda i: (i, 0)),
          pl.BlockSpec((_NBLK, OUT_CH), lambda i: (i, 0)),
      ],
      out_shape=[
          jax.ShapeDtypeStruct((N, OUT_CH), jnp.float32),
          jax.ShapeDtypeStruct((N, OUT_CH), jnp.float32),
      ],
  )(s2, q2, deg2, b2, Wb0)


def kernel(x, edge_index, W1, b1, W2, b2, Wb, bb):
  eidx = edge_index.reshape(2, NW, NCHUNK, CH)

  deg2 = _sc_deg(eidx)
  q1 = _tc_a(x, W1, deg2)
  s1 = _sc_spmm(q1, eidx, HID, 2)
  q2 = _tc_c(s1, q1, deg2, b1.reshape(1, HID), W2)
  s2 = _sc_spmm(q2, eidx, OUT_CH, 5)
  z, u = _tc_e(s2, q2, deg2, b2.reshape(1, OUT_CH), Wb[0])
  bb16 = jnp.broadcast_to(bb.reshape(1), (16,))
  return _sc_edge_decode(u, z, eidx, bb16).reshape(E, 1)


# speedup vs baseline: 30.9082x; 1.1601x over previous
import functools

import jax
import jax.numpy as jnp
from jax import lax
from jax.experimental import pallas as pl
from jax.experimental.pallas import tpu as pltpu
from jax.experimental.pallas import tpu_sc as plsc

N = 10000
E = 320000
IN_CH = 128
HID = 128
OUT_CH = 64

NC = 2
NS = 16
NW = NC * NS
EPW = E // NW
CH = 80
NCHUNK = EPW // CH
ROWB = 80
NROWCH = N // ROWB

_MESH = plsc.VectorSubcoreMesh(
    core_axis_name="c", subcore_axis_name="s", num_cores=NC, num_subcores=NS)


def _fill(buf, rows, width, value):
  vec = jnp.full((16,), value, jnp.float32)

  def body(r, carry):
    for j in range(width // 16):
      buf[r, pl.ds(j * 16, 16)] = vec
    return carry

  lax.fori_loop(0, rows, body, 0)


_DEGW = 8


def _sc_deg(eidx):
  @functools.partial(
      pl.kernel,
      out_type=jax.ShapeDtypeStruct((NC, N, 16), jnp.float32),
      mesh=_MESH,
      compiler_params=pltpu.CompilerParams(use_tc_tiling_on_sc=False),
      scratch_types=[
          pltpu.VMEM((EPW,), jnp.int32),
          pltpu.VMEM((ROWB, 16), jnp.float32),
          pltpu.VMEM((ROWB, 16), jnp.float32),
          pltpu.VMEM_SHARED((N, 16), jnp.float32),
          pltpu.SemaphoreType.DMA,
          pltpu.SemaphoreType.DMA,
      ],
  )
  def k(e_hbm, out_hbm, idxd, zrows, ones, acc, psem, ssem):
    c = lax.axis_index("c")
    s = lax.axis_index("s")
    gid = c * NS + s

    cp_idx = pltpu.async_copy(e_hbm.at[1, gid], idxd, psem)
    _fill(zrows, ROWB, 16, 0.0)
    _fill(ones, ROWB, 16, 1.0)

    def zacc(j, carry):
      @pl.when(lax.rem(j, NS) == s)
      def _():
        pltpu.sync_copy(zrows, acc.at[pl.ds(j * ROWB, ROWB)])
      return carry

    lax.fori_loop(0, NROWCH, zacc, 0)
    cp_idx.wait()
    plsc.subcore_barrier()

    def step(j, carry):
      jj = pl.ds(j * CH, CH)
      pltpu.async_copy(ones, acc.at[idxd.at[jj]], ssem, add=True)

      @pl.when(j >= _DEGW)
      def _():
        pltpu.make_async_copy(ones, acc.at[idxd.at[jj]], ssem).wait()
      return carry

    lax.fori_loop(0, NCHUNK, step, 0)

    def drain(j, carry):
      pltpu.make_async_copy(ones, acc.at[idxd.at[pl.ds(0, CH)]], ssem).wait()
      return carry

    lax.fori_loop(0, _DEGW, drain, 0)
    plsc.subcore_barrier()

    def cpout(j, carry):
      @pl.when(lax.rem(j, NS) == s)
      def _():
        pltpu.sync_copy(acc.at[pl.ds(j * ROWB, ROWB)], zrows)
        pltpu.sync_copy(zrows, out_hbm.at[c, pl.ds(j * ROWB, ROWB)])
      return carry

    lax.fori_loop(0, NROWCH, cpout, 0)

  return k(eidx)


def _sc_spmm(table, eidx, width, nb, ch):
  nch = EPW // ch

  @functools.partial(
      pl.kernel,
      out_type=jax.ShapeDtypeStruct((NC, N, width), jnp.float32),
      mesh=_MESH,
      compiler_params=pltpu.CompilerParams(use_tc_tiling_on_sc=False),
      scratch_types=[
          pltpu.VMEM((EPW,), jnp.int32),
          pltpu.VMEM((EPW,), jnp.int32),
      ] + [pltpu.VMEM((ch, width), jnp.float32)] * nb + [
          pltpu.VMEM_SHARED((N, width), jnp.float32),
          pltpu.SemaphoreType.DMA,
      ] + [pltpu.SemaphoreType.DMA] * (2 * nb),
  )
  def k(table_hbm, e_hbm, out_hbm, idxs, idxd, *bufs):
    rows = bufs[:nb]
    acc = bufs[nb]
    psem = bufs[nb + 1]
    g = bufs[nb + 2:nb + 2 + nb]
    st = bufs[nb + 2 + nb:]
    c = lax.axis_index("c")
    s = lax.axis_index("s")
    gid = c * NS + s

    cp_si = pltpu.async_copy(e_hbm.at[0, gid], idxs, psem)
    cp_di = pltpu.async_copy(e_hbm.at[1, gid], idxd, psem)

    _fill(rows[0], ch, width, 0.0)

    def zacc(j, carry):
      @pl.when(lax.rem(j, NS) == s)
      def _():
        pltpu.sync_copy(rows[0], acc.at[pl.ds(j * ch, ch)])
      return carry

    lax.fori_loop(0, N // ch, zacc, 0)
    cp_si.wait()
    cp_di.wait()
    for b in range(nb - 1):
      pltpu.async_copy(table_hbm.at[idxs.at[pl.ds(b * ch, ch)]], rows[b], g[b])
    plsc.subcore_barrier()

    def step(i, carry):
      for b in range(nb):
        j = nb * i + b
        js = pl.ds(j * ch, ch)
        ja = pl.ds((j + nb - 1) * ch, ch)
        pltpu.make_async_copy(table_hbm.at[idxs.at[js]], rows[b], g[b]).wait()
        bn = (b + nb - 1) % nb
        if b == 0:
          @pl.when(i > 0)
          def _():
            pltpu.make_async_copy(rows[bn], acc.at[idxd.at[js]], st[bn]).wait()
          pltpu.async_copy(table_hbm.at[idxs.at[ja]], rows[bn], g[bn])
        else:
          pltpu.make_async_copy(rows[bn], acc.at[idxd.at[js]], st[bn]).wait()

          @pl.when(i < nch // nb - 1)
          def _():
            pltpu.async_copy(table_hbm.at[idxs.at[ja]], rows[bn], g[bn])
        pltpu.async_copy(rows[b], acc.at[idxd.at[js]], st[b], add=True)
      return carry

    lax.fori_loop(0, nch // nb, step, 0)
    pltpu.make_async_copy(rows[nb - 1], acc.at[idxd.at[pl.ds(0, ch)]],
                          st[nb - 1]).wait()
    plsc.subcore_barrier()

    def cpout(j, carry):
      @pl.when(lax.rem(j, NS) == s)
      def _():
        pltpu.sync_copy(acc.at[pl.ds(j * ch, ch)], rows[0])
        pltpu.sync_copy(rows[0], out_hbm.at[c, pl.ds(j * ch, ch)])
      return carry

    lax.fori_loop(0, N // ch, cpout, 0)

  return k(table, eidx)


_NQ = OUT_CH // 16


def _sc_edge_decode(u, z, eidx, bb16):
  @functools.partial(
      pl.kernel,
      out_type=jax.ShapeDtypeStruct((E,), jnp.float32),
      mesh=_MESH,
      compiler_params=pltpu.CompilerParams(
          use_tc_tiling_on_sc=False, needs_layout_passes=False),
      scratch_types=[
          pltpu.VMEM((EPW,), jnp.int32),
          pltpu.VMEM((EPW,), jnp.int32),
          pltpu.VMEM((CH, OUT_CH), jnp.float32),
          pltpu.VMEM((CH, OUT_CH), jnp.float32),
          pltpu.VMEM((CH, OUT_CH), jnp.float32),
          pltpu.VMEM((CH, OUT_CH), jnp.float32),
          pltpu.VMEM((CH,), jnp.float32),
          pltpu.VMEM((CH,), jnp.float32),
          pltpu.VMEM((16,), jnp.float32),
      ] + [pltpu.SemaphoreType.DMA] * 7,
  )
  def k(u_hbm, z_hbm, e_hbm, bb_hbm, out_hbm,
        idxs, idxd, ubuf0, zbuf0, ubuf1, zbuf1, obuf0, obuf1, bbv,
        psem, gu0, gz0, gu1, gz1, wo0, wo1):
    c = lax.axis_index("c")
    s = lax.axis_index("s")
    gid = c * NS + s
    base = gid * EPW

    cp_si = pltpu.async_copy(e_hbm.at[0, gid], idxs, psem)
    cp_di = pltpu.async_copy(e_hbm.at[1, gid], idxd, psem)
    pltpu.sync_copy(bb_hbm, bbv)
    bias = bbv[...]
    lane = lax.iota(jnp.int32, 16)
    cp_si.wait()
    cp_di.wait()
    pltpu.async_copy(u_hbm.at[idxs.at[pl.ds(0, CH)]], ubuf0, gu0)
    pltpu.async_copy(z_hbm.at[idxd.at[pl.ds(0, CH)]], zbuf0, gz0)

    def dot_chunk(ubuf, zbuf, obuf):
      def grp(g, carry):
        res = jnp.zeros((16,), jnp.float32)
        for e in range(16):
          row = g * 16 + e
          acc = ubuf[row, pl.ds(0, 16)] * zbuf[row, pl.ds(0, 16)]
          for q in range(1, _NQ):
            acc = acc + ubuf[row, pl.ds(q * 16, 16)] * zbuf[row, pl.ds(q * 16, 16)]
          res = jnp.where(lane == e, jnp.full((16,), jnp.sum(acc)), res)
        obuf[pl.ds(g * 16, 16)] = 1.0 / (1.0 + jnp.exp(-(res + bias)))
        return carry

      lax.fori_loop(0, CH // 16, grp, 0)

    def step(i, carry):
      j0 = 2 * i
      j1 = 2 * i + 1
      off0 = pl.multiple_of(base + j0 * CH, 8)
      off1 = pl.multiple_of(base + j1 * CH, 8)
      j0s = pl.ds(j0 * CH, CH)
      j1s = pl.ds(j1 * CH, CH)
      j2s = pl.ds((j0 + 2) * CH, CH)
      pltpu.make_async_copy(u_hbm.at[idxs.at[j0s]], ubuf0, gu0).wait()
      pltpu.make_async_copy(z_hbm.at[idxd.at[j0s]], zbuf0, gz0).wait()
      pltpu.async_copy(u_hbm.at[idxs.at[j1s]], ubuf1, gu1)
      pltpu.async_copy(z_hbm.at[idxd.at[j1s]], zbuf1, gz1)

      @pl.when(i > 0)
      def _():
        pltpu.make_async_copy(obuf0, out_hbm.at[pl.ds(off0, CH)], wo0).wait()

      dot_chunk(ubuf0, zbuf0, obuf0)
      pltpu.async_copy(obuf0, out_hbm.at[pl.ds(off0, CH)], wo0)
      pltpu.make_async_copy(u_hbm.at[idxs.at[j1s]], ubuf1, gu1).wait()
      pltpu.make_async_copy(z_hbm.at[idxd.at[j1s]], zbuf1, gz1).wait()
      pltpu.async_copy(u_hbm.at[idxs.at[j2s]], ubuf0, gu0)
      pltpu.async_copy(z_hbm.at[idxd.at[j2s]], zbuf0, gz0)

      @pl.when(i > 0)
      def _():
        pltpu.make_async_copy(obuf1, out_hbm.at[pl.ds(off1, CH)], wo1).wait()

      dot_chunk(ubuf1, zbuf1, obuf1)
      pltpu.async_copy(obuf1, out_hbm.at[pl.ds(off1, CH)], wo1)
      return carry

    lax.fori_loop(0, NCHUNK // 2, step, 0)

    last = NCHUNK - 1
    offl = pl.multiple_of(base + last * CH, 8)
    lasts = pl.ds(last * CH, CH)
    pltpu.make_async_copy(u_hbm.at[idxs.at[lasts]], ubuf0, gu0).wait()
    pltpu.make_async_copy(z_hbm.at[idxd.at[lasts]], zbuf0, gz0).wait()
    pltpu.make_async_copy(obuf0, out_hbm.at[pl.ds(offl, CH)], wo0).wait()
    dot_chunk(ubuf0, zbuf0, obuf0)
    pltpu.sync_copy(obuf0, out_hbm.at[pl.ds(offl, CH)])
    pltpu.make_async_copy(obuf1, out_hbm.at[pl.ds(offl, CH)], wo1).wait()

  return k(u, z, eidx, bb16)


_NBLK = 2000
_GRID = N // _NBLK


def _dinv_from(deg_ref):
  deg = 1.0 + deg_ref[0, :, 0:1] + deg_ref[1, :, 0:1]
  return lax.rsqrt(deg)


_DEGSPEC = pl.BlockSpec((NC, _NBLK, 16), lambda i: (0, i, 0))


def _tc_a(x, W1, deg2):
  def body(x_ref, w_ref, d_ref, q_ref):
    dinv = _dinv_from(d_ref)
    p = jnp.dot(x_ref[:, :], w_ref[:, :], preferred_element_type=jnp.float32)
    q_ref[:, :] = dinv * p

  return pl.pallas_call(
      body,
      grid=(_GRID,),
      in_specs=[
          pl.BlockSpec((_NBLK, IN_CH), lambda i: (i, 0)),
          pl.BlockSpec((IN_CH, HID), lambda i: (0, 0)),
          _DEGSPEC,
      ],
      out_specs=pl.BlockSpec((_NBLK, HID), lambda i: (i, 0)),
      out_shape=jax.ShapeDtypeStruct((N, HID), jnp.float32),
  )(x, W1, deg2)


def _tc_c(s1, q1, deg2, b1, W2):
  def body(s_ref, q_ref, d_ref, bias_ref, w_ref, out_ref):
    dinv = _dinv_from(d_ref)
    h = dinv * (s_ref[0] + s_ref[1] + q_ref[:, :]) + bias_ref[:, :]
    h = jnp.maximum(h, 0.0)
    p2 = jnp.dot(h, w_ref[:, :], preferred_element_type=jnp.float32)
    out_ref[:, :] = dinv * p2

  return pl.pallas_call(
      body,
      grid=(_GRID,),
      in_specs=[
          pl.BlockSpec((NC, _NBLK, HID), lambda i: (0, i, 0)),
          pl.BlockSpec((_NBLK, HID), lambda i: (i, 0)),
          _DEGSPEC,
          pl.BlockSpec((1, HID), lambda i: (0, 0)),
          pl.BlockSpec((HID, OUT_CH), lambda i: (0, 0)),
      ],
      out_specs=pl.BlockSpec((_NBLK, OUT_CH), lambda i: (i, 0)),
      out_shape=jax.ShapeDtypeStruct((N, OUT_CH), jnp.float32),
  )(s1, q1, deg2, b1, W2)


def _tc_e(s2, q2, deg2, b2, Wb0):
  def body(s_ref, q_ref, d_ref, bias_ref, w_ref, z_ref, u_ref):
    dinv = _dinv_from(d_ref)
    z = dinv * (s_ref[0] + s_ref[1] + q_ref[:, :]) + bias_ref[:, :]
    z_ref[:, :] = z
    u_ref[:, :] = jnp.dot(z, w_ref[:, :], preferred_element_type=jnp.float32)

  return pl.pallas_call(
      body,
      grid=(_GRID,),
      in_specs=[
          pl.BlockSpec((NC, _NBLK, OUT_CH), lambda i: (0, i, 0)),
          pl.BlockSpec((_NBLK, OUT_CH), lambda i: (i, 0)),
          _DEGSPEC,
          pl.BlockSpec((1, OUT_CH), lambda i: (0, 0)),
          pl.BlockSpec((OUT_CH, OUT_CH), lambda i: (0, 0)),
      ],
      out_specs=[
          pl.BlockSpec((_NBLK, OUT_CH), lambda i: (i, 0)),
          pl.BlockSpec((_NBLK, OUT_CH), lambda i: (i, 0)),
      ],
      out_shape=[
          jax.ShapeDtypeStruct((N, OUT_CH), jnp.float32),
          jax.ShapeDtypeStruct((N, OUT_CH), jnp.float32),
      ],
  )(s2, q2, deg2, b2, Wb0)


def kernel(x, edge_index, W1, b1, W2, b2, Wb, bb):
  eidx = edge_index.reshape(2, NW, EPW)

  deg2 = _sc_deg(eidx)
  q1 = _tc_a(x, W1, deg2)
  s1 = _sc_spmm(q1, eidx, HID, 5, 40)
  q2 = _tc_c(s1, q1, deg2, b1.reshape(1, HID), W2)
  s2 = _sc_spmm(q2, eidx, OUT_CH, 5, 80)
  z, u = _tc_e(s2, q2, deg2, b2.reshape(1, OUT_CH), Wb[0])
  bb16 = jnp.broadcast_to(bb.reshape(1), (16,))
  return _sc_edge_decode(u, z, eidx, bb16).reshape(E, 1)
